# Initial kernel scaffold; baseline (speedup 1.0000x reference)
import functools

import jax
import jax.numpy as jnp
from jax import lax
from jax.experimental import pallas as pl
from jax.experimental.pallas import tpu as pltpu
from jax.experimental.pallas import tpu_sc as plsc

N = 10000
D = 128
H = 4
C = 32
E = 320000

NPAD = 10112
RPT = 632
ET = E + N
ETPAD = 330240
EA = ETPAD // 32
EC = ETPAD // 16
KA = 240
KC = 240
BLK = 632

_mesh = plsc.VectorSubcoreMesh(core_axis_name="c", subcore_axis_name="s")


def _pre_body(x_ref, wgat_ref, wgcn_ref, acomb_ref, xw_ref, xg_ref, asd_ref):
    x = x_ref[...]
    xw = jnp.dot(x, wgat_ref[...], preferred_element_type=jnp.float32)
    xg = jnp.dot(x, wgcn_ref[...], preferred_element_type=jnp.float32)
    xw_ref[...] = xw
    xg_ref[...] = xg
    asd_ref[...] = jnp.dot(xw, acomb_ref[...], preferred_element_type=jnp.float32)


def _tc_pre(xpad, W_gat, W_gcn, A_comb):
    grid = (NPAD // BLK,)
    return pl.pallas_call(
        _pre_body,
        grid=grid,
        in_specs=[
            pl.BlockSpec((BLK, D), lambda i: (i, 0)),
            pl.BlockSpec((D, D), lambda i: (0, 0)),
            pl.BlockSpec((D, D), lambda i: (0, 0)),
            pl.BlockSpec((D, 16), lambda i: (0, 0)),
        ],
        out_specs=[
            pl.BlockSpec((BLK, D), lambda i: (i, 0)),
            pl.BlockSpec((BLK, D), lambda i: (i, 0)),
            pl.BlockSpec((BLK, 16), lambda i: (i, 0)),
        ],
        out_shape=[
            jax.ShapeDtypeStruct((NPAD, D), jnp.float32),
            jax.ShapeDtypeStruct((NPAD, D), jnp.float32),
            jax.ShapeDtypeStruct((NPAD, 16), jnp.float32),
        ],
    )(xpad, W_gat, W_gcn, A_comb)


@functools.partial(
    pl.kernel,
    mesh=_mesh,
    out_type=jax.ShapeDtypeStruct((2, NPAD, 16), jnp.float32),
    scratch_types=[
        pltpu.VMEM((KA,), jnp.int32),
        pltpu.VMEM((KA,), jnp.int32),
        pltpu.VMEM((KA, 16), jnp.float32),
        pltpu.VMEM((KA, 16), jnp.float32),
        pltpu.VMEM((KA, 16), jnp.float32),
        pltpu.VMEM((RPT, 16), jnp.float32),
        pltpu.VMEM_SHARED((NPAD, 16), jnp.float32),
        pltpu.SemaphoreType.DMA,
        pltpu.SemaphoreType.DMA,
    ],
)
def _sc_a(src_hbm, dst_hbm, asd_hbm, out_hbm,
          sidx, didx, asrc, adst, evec, zbuf, acc, sem1, sem2):
    cid = lax.axis_index("c")
    sid = lax.axis_index("s")
    wid = sid * 2 + cid
    iota = lax.iota(jnp.int32, 16)
    zero16 = jnp.zeros((16,), jnp.float32)
    ecol4 = jnp.where(iota == 4, 1.0, 0.0).astype(jnp.float32)

    def _zrow(r, _):
        zbuf[r, :] = zero16
        return 0
    lax.fori_loop(0, RPT, _zrow, 0)

    def _erow(r, _):
        evec[r, :] = ecol4
        return 0
    lax.fori_loop(0, KA, _erow, 0)

    rbase = sid * RPT
    pltpu.sync_copy(zbuf, acc.at[pl.ds(rbase, RPT)])
    plsc.subcore_barrier()

    def _chunk(ch, _):
        base = pl.multiple_of(wid * EA + ch * KA, 8)
        pltpu.sync_copy(src_hbm.at[pl.ds(base, KA)], sidx)
        pltpu.sync_copy(dst_hbm.at[pl.ds(base, KA)], didx)
        cp1 = pltpu.async_copy(asd_hbm.at[sidx], asrc, sem1)
        cp2 = pltpu.async_copy(asd_hbm.at[didx], adst, sem2)
        cp1.wait()
        cp2.wait()
        for g in range(KA // 16):
            rows = g * 16 + iota
            for h in range(H):
                sa = plsc.load_gather(asrc, [rows, jnp.full((16,), h, jnp.int32)])
                da = plsc.load_gather(adst, [rows, jnp.full((16,), 4 + h, jnp.int32)])
                al = sa + da
                al = jnp.where(al > 0, al, 0.2 * al)
                ev = jnp.exp(al)
                plsc.store_scatter(evec, [rows, jnp.full((16,), h, jnp.int32)], ev)
        pltpu.sync_copy(evec, acc.at[didx], add=True)
        return 0
    lax.fori_loop(0, EA // KA, _chunk, 0)

    plsc.subcore_barrier()
    pltpu.sync_copy(acc.at[pl.ds(rbase, RPT)], out_hbm.at[cid, pl.ds(rbase, RPT)])


def _mid_body(a0_ref, a1_ref, asd_ref, keep_ref, s16_ref, d16_ref, ns_ref):
    den = a0_ref[...] + a1_ref[...]
    rden = 1.0 / (den + 1e-16)
    dis = jnp.where(den > 0, lax.rsqrt(jnp.maximum(den, 1e-30)), 0.0)
    ns = asd_ref[...] * keep_ref[...]
    ns = ns + jnp.dot(rden, s16_ref[...], preferred_element_type=jnp.float32)
    ns = ns + jnp.dot(dis, d16_ref[...], preferred_element_type=jnp.float32)
    ns_ref[...] = ns


def _tc_mid(a0, a1, asd, keep, s16, d16):
    grid = (NPAD // BLK,)
    return pl.pallas_call(
        _mid_body,
        grid=grid,
        in_specs=[
            pl.BlockSpec((BLK, 16), lambda i: (i, 0)),
            pl.BlockSpec((BLK, 16), lambda i: (i, 0)),
            pl.BlockSpec((BLK, 16), lambda i: (i, 0)),
            pl.BlockSpec((1, 16), lambda i: (0, 0)),
            pl.BlockSpec((16, 16), lambda i: (0, 0)),
            pl.BlockSpec((16, 16), lambda i: (0, 0)),
        ],
        out_specs=pl.BlockSpec((BLK, 16), lambda i: (i, 0)),
        out_shape=jax.ShapeDtypeStruct((NPAD, 16), jnp.float32),
    )(a0, a1, asd, keep, s16, d16)


@functools.partial(
    pl.kernel,
    mesh=_mesh,
    out_type=jax.ShapeDtypeStruct((2, NPAD, D), jnp.float32),
    scratch_types=[
        pltpu.VMEM((KC,), jnp.int32),
        pltpu.VMEM((KC,), jnp.int32),
        pltpu.VMEM((KC,), jnp.int32),
        pltpu.VMEM((KC,), jnp.int32),
        pltpu.VMEM((KC, 16), jnp.float32),
        pltpu.VMEM((KC, 16), jnp.float32),
        pltpu.VMEM((KC, 16), jnp.float32),
        pltpu.VMEM((KC,), jnp.float32),
        pltpu.VMEM((KC,), jnp.float32),
        pltpu.VMEM((KC, D), jnp.float32),
        pltpu.VMEM((RPT // 4, D), jnp.float32),
        pltpu.VMEM_SHARED((NPAD, D), jnp.float32),
        pltpu.VMEM_SHARED((NPAD * 16,), jnp.float32),
        pltpu.SemaphoreType.DMA,
        pltpu.SemaphoreType.DMA,
        pltpu.SemaphoreType.DMA,
    ],
)
def _sc_c(src_hbm, dst_hbm, ns_hbm, nsflat_hbm, xw_hbm, xg_hbm, out_hbm,
          sidx, didx, sidx2, didx2, ssml, dsml, wbuf, sdis, ddis, feats, zbuf,
          acc, nstab, sem1, sem2, sem3):
    cid = lax.axis_index("c")
    sid = lax.axis_index("s")
    iota = lax.iota(jnp.int32, 16)
    zero16 = jnp.zeros((16,), jnp.float32)

    def _zrow(r, _):
        for v in range(D // 16):
            zbuf[r, pl.ds(v * 16, 16)] = zero16
        return 0
    lax.fori_loop(0, RPT // 4, _zrow, 0)
    rbase = sid * RPT
    for q in range(4):
        pltpu.sync_copy(zbuf, acc.at[pl.ds(rbase + q * (RPT // 4), RPT // 4)])

    @pl.when(jnp.logical_and(cid == 1, sid == 0))
    def _stage():
        pltpu.sync_copy(nsflat_hbm, nstab)

    plsc.subcore_barrier()

    ebase0 = sid * EC

    @pl.when(cid == 0)
    def _gat_core():
        def _chunk(ch, _):
            base = pl.multiple_of(ebase0 + ch * KC, 8)
            pltpu.sync_copy(src_hbm.at[pl.ds(base, KC)], sidx)
            pltpu.sync_copy(dst_hbm.at[pl.ds(base, KC)], didx)
            cp1 = pltpu.async_copy(ns_hbm.at[sidx], ssml, sem1)
            cp2 = pltpu.async_copy(ns_hbm.at[didx], dsml, sem2)
            cp3 = pltpu.async_copy(xw_hbm.at[sidx], feats, sem3)
            cp1.wait()
            cp2.wait()
            cp3.wait()
            for g in range(KC // 16):
                rows = g * 16 + iota
                for h in range(H):
                    sa = plsc.load_gather(ssml, [rows, jnp.full((16,), h, jnp.int32)])
                    da = plsc.load_gather(dsml, [rows, jnp.full((16,), 4 + h, jnp.int32)])
                    rd = plsc.load_gather(dsml, [rows, jnp.full((16,), 8 + h, jnp.int32)])
                    al = sa + da
                    al = jnp.where(al > 0, al, 0.2 * al)
                    w = jnp.exp(al) * rd
                    plsc.store_scatter(wbuf, [rows, jnp.full((16,), h, jnp.int32)], w)

            def _edge(j, _):
                j16 = jnp.full((16,), j, jnp.int32)
                for h in range(H):
                    wsp = plsc.load_gather(wbuf, [j16, jnp.full((16,), h, jnp.int32)])
                    for half in range(2):
                        v = 2 * h + half
                        feats[j, pl.ds(v * 16, 16)] = feats[j, pl.ds(v * 16, 16)] * wsp
                return 0
            lax.fori_loop(0, KC, _edge, 0)
            pltpu.sync_copy(feats, acc.at[didx], add=True)
            return 0
        lax.fori_loop(0, EC // KC, _chunk, 0)

    @pl.when(cid == 1)
    def _gcn_core():
        def _chunk(ch, _):
            base = pl.multiple_of(ebase0 + ch * KC, 8)
            pltpu.sync_copy(src_hbm.at[pl.ds(base, KC)], sidx)
            pltpu.sync_copy(dst_hbm.at[pl.ds(base, KC)], didx)
            for g in range(KC // 16):
                sl = pl.ds(g * 16, 16)
                sidx2[sl] = sidx[sl] * 16 + 12
                didx2[sl] = didx[sl] * 16 + 12
            cp1 = pltpu.async_copy(nstab.at[sidx2], sdis, sem1)
            cp2 = pltpu.async_copy(nstab.at[didx2], ddis, sem2)
            cp3 = pltpu.async_copy(xg_hbm.at[sidx], feats, sem3)
            cp1.wait()
            cp2.wait()
            for g in range(KC // 16):
                sl = pl.ds(g * 16, 16)
                nrm = sdis[sl] * ddis[sl]
                sdis[sl] = nrm
            cp3.wait()

            def _edge(j, _):
                j16 = jnp.full((16,), j, jnp.int32)
                nsp = plsc.load_gather(sdis, [j16])
                for v in range(D // 16):
                    feats[j, pl.ds(v * 16, 16)] = feats[j, pl.ds(v * 16, 16)] * nsp
                return 0
            lax.fori_loop(0, KC, _edge, 0)
            pltpu.sync_copy(feats, acc.at[didx], add=True)
            return 0
        lax.fori_loop(0, EC // KC, _chunk, 0)

    plsc.subcore_barrier()
    pltpu.sync_copy(acc.at[pl.ds(rbase, RPT)], out_hbm.at[cid, pl.ds(rbase, RPT)])


def _post_body(gat_ref, gcn_ref, x_ref, bgat_ref, bgcn_ref, wga_ref, wgb_ref,
               bgate_ref, gamma_ref, beta_ref, o_ref):
    gat = gat_ref[...] + bgat_ref[...]
    gcn = gcn_ref[...] + bgcn_ref[...]
    lg = (jnp.dot(gat, wga_ref[...], preferred_element_type=jnp.float32)
          + jnp.dot(gcn, wgb_ref[...], preferred_element_type=jnp.float32)
          + bgate_ref[...])
    m = jnp.max(lg, axis=-1, keepdims=True)
    eg = jnp.exp(lg - m)
    sm = eg / jnp.sum(eg, axis=-1, keepdims=True)
    out = sm[:, 0:1] * gat + sm[:, 1:2] * gcn
    y = out + x_ref[...]
    mu = jnp.mean(y, axis=-1, keepdims=True)
    yc = y - mu
    var = jnp.mean(yc * yc, axis=-1, keepdims=True)
    o_ref[...] = gamma_ref[...] * yc * lax.rsqrt(var + 1e-5) + beta_ref[...]


def _tc_post(gat_raw, gcn_raw, xpad, b_gat, b_gcn, wga, wgb, b_gate, gamma, beta):
    grid = (NPAD // BLK,)
    return pl.pallas_call(
        _post_body,
        grid=grid,
        in_specs=[
            pl.BlockSpec((BLK, D), lambda i: (i, 0)),
            pl.BlockSpec((BLK, D), lambda i: (i, 0)),
            pl.BlockSpec((BLK, D), lambda i: (i, 0)),
            pl.BlockSpec((1, D), lambda i: (0, 0)),
            pl.BlockSpec((1, D), lambda i: (0, 0)),
            pl.BlockSpec((D, 2), lambda i: (0, 0)),
            pl.BlockSpec((D, 2), lambda i: (0, 0)),
            pl.BlockSpec((1, 2), lambda i: (0, 0)),
            pl.BlockSpec((1, D), lambda i: (0, 0)),
            pl.BlockSpec((1, D), lambda i: (0, 0)),
        ],
        out_specs=pl.BlockSpec((BLK, D), lambda i: (i, 0)),
        out_shape=jax.ShapeDtypeStruct((NPAD, D), jnp.float32),
    )(gat_raw, gcn_raw, xpad, b_gat, b_gcn, wga, wgb, b_gate, gamma, beta)


def kernel(x, edge_index, W_gat, att_src, att_dst, b_gat, W_gcn, b_gcn,
           W_gate, b_gate, gamma, beta):
    xpad = jnp.pad(x, ((0, NPAD - N), (0, 0)))
    loops = jnp.arange(N, dtype=jnp.int32)
    src = jnp.concatenate([edge_index[0], loops,
                           jnp.zeros((ETPAD - ET,), jnp.int32)])
    dst = jnp.concatenate([edge_index[1], loops,
                           jnp.full((ETPAD - ET,), N, jnp.int32)])

    eyeH = jnp.eye(H, dtype=jnp.float32)
    A_src = (att_src[:, :, None] * eyeH[:, None, :]).reshape(D, H)
    A_dst = (att_dst[:, :, None] * eyeH[:, None, :]).reshape(D, H)
    A_comb = jnp.concatenate([A_src, A_dst, jnp.zeros((D, 8), jnp.float32)], axis=1)

    xw, xg, asd = _tc_pre(xpad, W_gat, W_gcn, A_comb)

    accA = _sc_a(src, dst, asd)

    col = jnp.arange(16)
    keep = (col < 8).astype(jnp.float32)[None, :]
    s16 = ((col[:, None] + 8 == col[None, :]) & (col[:, None] < 4)).astype(jnp.float32)
    d16 = ((col[:, None] == 4) & (col[None, :] == 12)).astype(jnp.float32)
    nsmall = _tc_mid(accA[0], accA[1], asd, keep, s16, d16)

    raw = _sc_c(src, dst, nsmall, nsmall.reshape(-1), xw, xg)

    y = _tc_post(raw[0], raw[1], xpad, b_gat[None, :], b_gcn[None, :],
                 W_gate[:D], W_gate[D:], b_gate[None, :], gamma[None, :],
                 beta[None, :])
    return y[:N]


# speedup vs baseline: 30.9317x; 30.9317x over previous
import functools

import jax
import jax.numpy as jnp
from jax import lax
from jax.experimental import pallas as pl
from jax.experimental.pallas import tpu as pltpu
from jax.experimental.pallas import tpu_sc as plsc

N = 10000
D = 128
H = 4
C = 32
E = 320000

NPAD = 10112
RPT = 632
ET = E + N
ETPAD = 330240
EA = ETPAD // 32
EC = ETPAD // 16
KA = 240
KC = 240
BLK = 632

_SC_CACHE = {}


def _pre_body(x_ref, wgat_ref, wgcn_ref, acomb_ref,
              xwlo_ref, xwhi_ref, xglo_ref, xghi_ref, asd_ref):
    x = x_ref[...]
    xw = jnp.dot(x, wgat_ref[...], preferred_element_type=jnp.float32)
    xg = jnp.dot(x, wgcn_ref[...], preferred_element_type=jnp.float32)
    xwlo_ref[...] = xw[:, :64]
    xwhi_ref[...] = xw[:, 64:]
    xglo_ref[...] = xg[:, :64]
    xghi_ref[...] = xg[:, 64:]
    asd_ref[...] = jnp.dot(xw, acomb_ref[...], preferred_element_type=jnp.float32)


def _tc_pre(xpad, W_gat, W_gcn, A_comb):
    grid = (NPAD // BLK,)
    return pl.pallas_call(
        _pre_body,
        grid=grid,
        in_specs=[
            pl.BlockSpec((BLK, D), lambda i: (i, 0)),
            pl.BlockSpec((D, D), lambda i: (0, 0)),
            pl.BlockSpec((D, D), lambda i: (0, 0)),
            pl.BlockSpec((D, 16), lambda i: (0, 0)),
        ],
        out_specs=[
            pl.BlockSpec((BLK, 64), lambda i: (i, 0)),
            pl.BlockSpec((BLK, 64), lambda i: (i, 0)),
            pl.BlockSpec((BLK, 64), lambda i: (i, 0)),
            pl.BlockSpec((BLK, 64), lambda i: (i, 0)),
            pl.BlockSpec((BLK, 16), lambda i: (i, 0)),
        ],
        out_shape=[
            jax.ShapeDtypeStruct((NPAD, 64), jnp.float32),
            jax.ShapeDtypeStruct((NPAD, 64), jnp.float32),
            jax.ShapeDtypeStruct((NPAD, 64), jnp.float32),
            jax.ShapeDtypeStruct((NPAD, 64), jnp.float32),
            jax.ShapeDtypeStruct((NPAD, 16), jnp.float32),
        ],
    )(xpad, W_gat, W_gcn, A_comb)


def _sc_a_body(src_hbm, dst_hbm, asd_hbm, out_hbm,
          sidx, didx, asrc, adst, evec, zbuf, acc, sem1, sem2):
    cid = lax.axis_index("c")
    sid = lax.axis_index("s")
    wid = sid * 2 + cid
    iota = lax.iota(jnp.int32, 16)
    zero16 = jnp.zeros((16,), jnp.float32)
    ecol4 = jnp.where(iota == 4, 1.0, 0.0).astype(jnp.float32)

    def _zrow(r, _):
        zbuf[r, :] = zero16
        return 0
    lax.fori_loop(0, RPT, _zrow, 0)

    def _erow(r, _):
        evec[r, :] = ecol4
        return 0
    lax.fori_loop(0, KA, _erow, 0)

    rbase = sid * RPT
    pltpu.sync_copy(zbuf, acc.at[pl.ds(rbase, RPT)])
    plsc.subcore_barrier()

    def _chunk(ch, _):
        base = pl.multiple_of(wid * EA + ch * KA, 8)
        pltpu.sync_copy(src_hbm.at[pl.ds(base, KA)], sidx)
        pltpu.sync_copy(dst_hbm.at[pl.ds(base, KA)], didx)
        cp1 = pltpu.async_copy(asd_hbm.at[sidx], asrc, sem1)
        cp2 = pltpu.async_copy(asd_hbm.at[didx], adst, sem2)
        cp1.wait()
        cp2.wait()
        for g in range(KA // 16):
            rows = g * 16 + iota
            for h in range(H):
                sa = plsc.load_gather(asrc, [rows, jnp.full((16,), h, jnp.int32)])
                da = plsc.load_gather(adst, [rows, jnp.full((16,), 4 + h, jnp.int32)])
                al = sa + da
                al = jnp.where(al > 0, al, 0.2 * al)
                ev = jnp.exp(al)
                plsc.store_scatter(evec, [rows, jnp.full((16,), h, jnp.int32)], ev)
        pltpu.sync_copy(evec, acc.at[didx], add=True)
        return 0
    lax.fori_loop(0, EA // KA, _chunk, 0)

    plsc.subcore_barrier()
    pltpu.sync_copy(acc.at[pl.ds(rbase, RPT)], out_hbm.at[cid, pl.ds(rbase, RPT)])


def _mid_body(a0_ref, a1_ref, asd_ref, keep_ref, s16_ref, d16_ref, ns_ref):
    den = a0_ref[...] + a1_ref[...]
    rden = 1.0 / (den + 1e-16)
    dis = jnp.where(den > 0, lax.rsqrt(jnp.maximum(den, 1e-30)), 0.0)
    ns = asd_ref[...] * keep_ref[...]
    ns = ns + jnp.dot(rden, s16_ref[...], preferred_element_type=jnp.float32)
    ns = ns + jnp.dot(dis, d16_ref[...], preferred_element_type=jnp.float32)
    ns_ref[...] = ns


def _tc_mid(a0, a1, asd, keep, s16, d16):
    grid = (NPAD // BLK,)
    return pl.pallas_call(
        _mid_body,
        grid=grid,
        in_specs=[
            pl.BlockSpec((BLK, 16), lambda i: (i, 0)),
            pl.BlockSpec((BLK, 16), lambda i: (i, 0)),
            pl.BlockSpec((BLK, 16), lambda i: (i, 0)),
            pl.BlockSpec((1, 16), lambda i: (0, 0)),
            pl.BlockSpec((16, 16), lambda i: (0, 0)),
            pl.BlockSpec((16, 16), lambda i: (0, 0)),
        ],
        out_specs=pl.BlockSpec((BLK, 16), lambda i: (i, 0)),
        out_shape=jax.ShapeDtypeStruct((NPAD, 16), jnp.float32),
    )(a0, a1, asd, keep, s16, d16)


def _sc_c_body(src_hbm, dst_hbm, ns_hbm, xwlo_hbm, xwhi_hbm, xglo_hbm, xghi_hbm,
               out_hbm, sidx, didx, ssml, dsml, wbuf, nrm, feats, zbuf,
               acc, sem1, sem2, sem3):
    cid = lax.axis_index("c")
    sid = lax.axis_index("s")
    iota = lax.iota(jnp.int32, 16)
    zero16 = jnp.zeros((16,), jnp.float32)
    rbase = sid * RPT
    ebase0 = sid * EC

    def _zero_acc():
        def _zrow(r, _):
            for v in range(64 // 16):
                zbuf[r, pl.ds(v * 16, 16)] = zero16
            return 0
        lax.fori_loop(0, RPT // 4, _zrow, 0)
        for q in range(4):
            pltpu.sync_copy(zbuf, acc.at[pl.ds(rbase + q * (RPT // 4), RPT // 4)])

    def _gat_half(tab_hbm, hf):
        def _chunk(ch, _):
            base = pl.multiple_of(ebase0 + ch * KC, 8)
            pltpu.sync_copy(src_hbm.at[pl.ds(base, KC)], sidx)
            pltpu.sync_copy(dst_hbm.at[pl.ds(base, KC)], didx)
            cp1 = pltpu.async_copy(ns_hbm.at[sidx], ssml, sem1)
            cp2 = pltpu.async_copy(ns_hbm.at[didx], dsml, sem2)
            cp3 = pltpu.async_copy(tab_hbm.at[sidx], feats, sem3)
            cp1.wait()
            cp2.wait()
            cp3.wait()
            for g in range(KC // 16):
                rows = g * 16 + iota
                for h in (2 * hf, 2 * hf + 1):
                    sa = plsc.load_gather(ssml, [rows, jnp.full((16,), h, jnp.int32)])
                    da = plsc.load_gather(dsml, [rows, jnp.full((16,), 4 + h, jnp.int32)])
                    rd = plsc.load_gather(dsml, [rows, jnp.full((16,), 8 + h, jnp.int32)])
                    al = sa + da
                    al = jnp.where(al > 0, al, 0.2 * al)
                    w = jnp.exp(al) * rd
                    plsc.store_scatter(wbuf, [rows, jnp.full((16,), h, jnp.int32)], w)

            def _edge(j, _):
                j16 = jnp.full((16,), j, jnp.int32)
                for hh in range(2):
                    wsp = plsc.load_gather(
                        wbuf, [j16, jnp.full((16,), 2 * hf + hh, jnp.int32)])
                    for half in range(2):
                        v = 2 * hh + half
                        feats[j, pl.ds(v * 16, 16)] = feats[j, pl.ds(v * 16, 16)] * wsp
                return 0
            lax.fori_loop(0, KC, _edge, 0)
            pltpu.sync_copy(feats, acc.at[didx], add=True)
            return 0
        lax.fori_loop(0, EC // KC, _chunk, 0)

    def _gcn_half(tab_hbm):
        def _chunk(ch, _):
            base = pl.multiple_of(ebase0 + ch * KC, 8)
            pltpu.sync_copy(src_hbm.at[pl.ds(base, KC)], sidx)
            pltpu.sync_copy(dst_hbm.at[pl.ds(base, KC)], didx)
            cp1 = pltpu.async_copy(ns_hbm.at[sidx], ssml, sem1)
            cp2 = pltpu.async_copy(ns_hbm.at[didx], dsml, sem2)
            cp3 = pltpu.async_copy(tab_hbm.at[sidx], feats, sem3)
            cp1.wait()
            cp2.wait()
            for g in range(KC // 16):
                rows = g * 16 + iota
                c12 = jnp.full((16,), 12, jnp.int32)
                sd = plsc.load_gather(ssml, [rows, c12])
                dd = plsc.load_gather(dsml, [rows, c12])
                nrm[pl.ds(g * 16, 16)] = sd * dd
            cp3.wait()

            def _edge(j, _):
                j16 = jnp.full((16,), j, jnp.int32)
                nsp = plsc.load_gather(nrm, [j16])
                for v in range(64 // 16):
                    feats[j, pl.ds(v * 16, 16)] = feats[j, pl.ds(v * 16, 16)] * nsp
                return 0
            lax.fori_loop(0, KC, _edge, 0)
            pltpu.sync_copy(feats, acc.at[didx], add=True)
            return 0
        lax.fori_loop(0, EC // KC, _chunk, 0)

    for hf in range(2):
        _zero_acc()
        plsc.subcore_barrier()

        @pl.when(cid == 0)
        def _gat_core(hf=hf):
            _gat_half(xwlo_hbm if hf == 0 else xwhi_hbm, hf)

        @pl.when(cid == 1)
        def _gcn_core(hf=hf):
            _gcn_half(xglo_hbm if hf == 0 else xghi_hbm)

        plsc.subcore_barrier()
        pltpu.sync_copy(acc.at[pl.ds(rbase, RPT)],
                        out_hbm.at[cid, hf, pl.ds(rbase, RPT)])


def _post_body(gatlo_ref, gathi_ref, gcnlo_ref, gcnhi_ref, x_ref, bgat_ref,
               bgcn_ref, wga_ref, wgb_ref, bgate_ref, gamma_ref, beta_ref,
               o_ref):
    gat = jnp.concatenate([gatlo_ref[...], gathi_ref[...]], axis=1) + bgat_ref[...]
    gcn = jnp.concatenate([gcnlo_ref[...], gcnhi_ref[...]], axis=1) + bgcn_ref[...]
    lg = (jnp.dot(gat, wga_ref[...], preferred_element_type=jnp.float32)
          + jnp.dot(gcn, wgb_ref[...], preferred_element_type=jnp.float32)
          + bgate_ref[...])
    m = jnp.max(lg, axis=-1, keepdims=True)
    eg = jnp.exp(lg - m)
    sm = eg / jnp.sum(eg, axis=-1, keepdims=True)
    out = sm[:, 0:1] * gat + sm[:, 1:2] * gcn
    y = out + x_ref[...]
    mu = jnp.mean(y, axis=-1, keepdims=True)
    yc = y - mu
    var = jnp.mean(yc * yc, axis=-1, keepdims=True)
    o_ref[...] = gamma_ref[...] * yc * lax.rsqrt(var + 1e-5) + beta_ref[...]


def _tc_post(gatlo, gathi, gcnlo, gcnhi, xpad, b_gat, b_gcn, wga, wgb,
             b_gate, gamma, beta):
    grid = (NPAD // BLK,)
    return pl.pallas_call(
        _post_body,
        grid=grid,
        in_specs=[
            pl.BlockSpec((BLK, 64), lambda i: (i, 0)),
            pl.BlockSpec((BLK, 64), lambda i: (i, 0)),
            pl.BlockSpec((BLK, 64), lambda i: (i, 0)),
            pl.BlockSpec((BLK, 64), lambda i: (i, 0)),
            pl.BlockSpec((BLK, D), lambda i: (i, 0)),
            pl.BlockSpec((1, D), lambda i: (0, 0)),
            pl.BlockSpec((1, D), lambda i: (0, 0)),
            pl.BlockSpec((D, 2), lambda i: (0, 0)),
            pl.BlockSpec((D, 2), lambda i: (0, 0)),
            pl.BlockSpec((1, 2), lambda i: (0, 0)),
            pl.BlockSpec((1, D), lambda i: (0, 0)),
            pl.BlockSpec((1, D), lambda i: (0, 0)),
        ],
        out_specs=pl.BlockSpec((BLK, D), lambda i: (i, 0)),
        out_shape=jax.ShapeDtypeStruct((NPAD, D), jnp.float32),
    )(gatlo, gathi, gcnlo, gcnhi, xpad, b_gat, b_gcn, wga, wgb, b_gate,
      gamma, beta)


def _sc_kernels():
    if "a" not in _SC_CACHE:
        mesh = plsc.VectorSubcoreMesh(core_axis_name="c", subcore_axis_name="s")
        _SC_CACHE["a"] = pl.kernel(
            _sc_a_body,
            mesh=mesh,
            compiler_params=pltpu.CompilerParams(
                needs_layout_passes=False, use_tc_tiling_on_sc=False),
            out_type=jax.ShapeDtypeStruct((2, NPAD, 16), jnp.float32),
            scratch_types=[
                pltpu.VMEM((KA,), jnp.int32),
                pltpu.VMEM((KA,), jnp.int32),
                pltpu.VMEM((KA, 16), jnp.float32),
                pltpu.VMEM((KA, 16), jnp.float32),
                pltpu.VMEM((KA, 16), jnp.float32),
                pltpu.VMEM((RPT, 16), jnp.float32),
                pltpu.VMEM_SHARED((NPAD, 16), jnp.float32),
                pltpu.SemaphoreType.DMA,
                pltpu.SemaphoreType.DMA,
            ],
        )
        _SC_CACHE["c"] = pl.kernel(
            _sc_c_body,
            mesh=mesh,
            compiler_params=pltpu.CompilerParams(
                needs_layout_passes=False, use_tc_tiling_on_sc=False),
            out_type=jax.ShapeDtypeStruct((2, 2, NPAD, 64), jnp.float32),
            scratch_types=[
                pltpu.VMEM((KC,), jnp.int32),
                pltpu.VMEM((KC,), jnp.int32),
                pltpu.VMEM((KC, 16), jnp.float32),
                pltpu.VMEM((KC, 16), jnp.float32),
                pltpu.VMEM((KC, 16), jnp.float32),
                pltpu.VMEM((KC,), jnp.float32),
                pltpu.VMEM((KC, 64), jnp.float32),
                pltpu.VMEM((RPT // 4, 64), jnp.float32),
                pltpu.VMEM_SHARED((NPAD, 64), jnp.float32),
                pltpu.SemaphoreType.DMA,
                pltpu.SemaphoreType.DMA,
                pltpu.SemaphoreType.DMA,
            ],
        )
    return _SC_CACHE["a"], _SC_CACHE["c"]


def kernel(x, edge_index, W_gat, att_src, att_dst, b_gat, W_gcn, b_gcn,
           W_gate, b_gate, gamma, beta):
    xpad = jnp.pad(x, ((0, NPAD - N), (0, 0)))
    loops = jnp.arange(N, dtype=jnp.int32)
    src = jnp.concatenate([edge_index[0], loops,
                           jnp.zeros((ETPAD - ET,), jnp.int32)])
    dst = jnp.concatenate([edge_index[1], loops,
                           jnp.full((ETPAD - ET,), N, jnp.int32)])

    eyeH = jnp.eye(H, dtype=jnp.float32)
    A_src = (att_src[:, :, None] * eyeH[:, None, :]).reshape(D, H)
    A_dst = (att_dst[:, :, None] * eyeH[:, None, :]).reshape(D, H)
    A_comb = jnp.concatenate([A_src, A_dst, jnp.zeros((D, 8), jnp.float32)], axis=1)

    xwlo, xwhi, xglo, xghi, asd = _tc_pre(xpad, W_gat, W_gcn, A_comb)

    sc_a, sc_c = _sc_kernels()
    accA = sc_a(src, dst, asd)

    col = jnp.arange(16)
    keep = (col < 8).astype(jnp.float32)[None, :]
    s16 = ((col[:, None] + 8 == col[None, :]) & (col[:, None] < 4)).astype(jnp.float32)
    d16 = ((col[:, None] == 4) & (col[None, :] == 12)).astype(jnp.float32)
    nsmall = _tc_mid(accA[0], accA[1], asd, keep, s16, d16)

    raw = sc_c(src, dst, nsmall, xwlo, xwhi, xglo, xghi)

    y = _tc_post(raw[0, 0], raw[0, 1], raw[1, 0], raw[1, 1], xpad,
                 b_gat[None, :], b_gcn[None, :], W_gate[:D], W_gate[D:],
                 b_gate[None, :], gamma[None, :], beta[None, :])
    return y[:N]


# speedup vs baseline: 31.1385x; 1.0067x over previous
import functools

import jax
import jax.numpy as jnp
from jax import lax
from jax.experimental import pallas as pl
from jax.experimental.pallas import tpu as pltpu
from jax.experimental.pallas import tpu_sc as plsc

N = 10000
D = 128
H = 4
C = 32
E = 320000

NPAD = 10112
RPT = 632
ET = E + N
ETPAD = 330240
EA = ETPAD // 32
EC = ETPAD // 16
KA = 240
KC = 240
BLK = 632

_SC_CACHE = {}


def _pre_body(x_ref, wgat_ref, wgcn_ref, acomb_ref,
              xwlo_ref, xwhi_ref, xglo_ref, xghi_ref, asd_ref):
    x = x_ref[...]
    xw = jnp.dot(x, wgat_ref[...], preferred_element_type=jnp.float32)
    xg = jnp.dot(x, wgcn_ref[...], preferred_element_type=jnp.float32)
    xwlo_ref[...] = xw[:, :64]
    xwhi_ref[...] = xw[:, 64:]
    xglo_ref[...] = xg[:, :64]
    xghi_ref[...] = xg[:, 64:]
    asd_ref[...] = jnp.dot(xw, acomb_ref[...], preferred_element_type=jnp.float32)


def _tc_pre(xpad, W_gat, W_gcn, A_comb):
    grid = (NPAD // BLK,)
    return pl.pallas_call(
        _pre_body,
        grid=grid,
        in_specs=[
            pl.BlockSpec((BLK, D), lambda i: (i, 0)),
            pl.BlockSpec((D, D), lambda i: (0, 0)),
            pl.BlockSpec((D, D), lambda i: (0, 0)),
            pl.BlockSpec((D, 16), lambda i: (0, 0)),
        ],
        out_specs=[
            pl.BlockSpec((BLK, 64), lambda i: (i, 0)),
            pl.BlockSpec((BLK, 64), lambda i: (i, 0)),
            pl.BlockSpec((BLK, 64), lambda i: (i, 0)),
            pl.BlockSpec((BLK, 64), lambda i: (i, 0)),
            pl.BlockSpec((BLK, 16), lambda i: (i, 0)),
        ],
        out_shape=[
            jax.ShapeDtypeStruct((NPAD, 64), jnp.float32),
            jax.ShapeDtypeStruct((NPAD, 64), jnp.float32),
            jax.ShapeDtypeStruct((NPAD, 64), jnp.float32),
            jax.ShapeDtypeStruct((NPAD, 64), jnp.float32),
            jax.ShapeDtypeStruct((NPAD, 16), jnp.float32),
        ],
    )(xpad, W_gat, W_gcn, A_comb)


def _sc_a_body(src_hbm, dst_hbm, asd_hbm, out_hbm,
          sidx, didx, asrc, adst, evec, zbuf, acc, sem1, sem2):
    cid = lax.axis_index("c")
    sid = lax.axis_index("s")
    wid = sid * 2 + cid
    iota = lax.iota(jnp.int32, 16)
    zero16 = jnp.zeros((16,), jnp.float32)
    ecol4 = jnp.where(iota == 4, 1.0, 0.0).astype(jnp.float32)

    def _zrow(r, _):
        zbuf[r, :] = zero16
        return 0
    lax.fori_loop(0, RPT, _zrow, 0)

    def _erow(r, _):
        evec[r, :] = ecol4
        return 0
    lax.fori_loop(0, KA, _erow, 0)

    rbase = sid * RPT
    pltpu.sync_copy(zbuf, acc.at[pl.ds(rbase, RPT)])
    plsc.subcore_barrier()

    def _chunk(ch, _):
        base = pl.multiple_of(wid * EA + ch * KA, 8)
        pltpu.sync_copy(src_hbm.at[pl.ds(base, KA)], sidx)
        pltpu.sync_copy(dst_hbm.at[pl.ds(base, KA)], didx)
        cp1 = pltpu.async_copy(asd_hbm.at[sidx], asrc, sem1)
        cp2 = pltpu.async_copy(asd_hbm.at[didx], adst, sem2)
        cp1.wait()
        cp2.wait()
        for g in range(KA // 16):
            rows = g * 16 + iota
            for h in range(H):
                sa = plsc.load_gather(asrc, [rows, jnp.full((16,), h, jnp.int32)])
                da = plsc.load_gather(adst, [rows, jnp.full((16,), 4 + h, jnp.int32)])
                al = sa + da
                al = jnp.where(al > 0, al, 0.2 * al)
                ev = jnp.exp(al)
                plsc.store_scatter(evec, [rows, jnp.full((16,), h, jnp.int32)], ev)
        pltpu.sync_copy(evec, acc.at[didx], add=True)
        return 0
    lax.fori_loop(0, EA // KA, _chunk, 0)

    plsc.subcore_barrier()
    pltpu.sync_copy(acc.at[pl.ds(rbase, RPT)], out_hbm.at[cid, pl.ds(rbase, RPT)])


def _mid_body(a0_ref, a1_ref, asd_ref, xglo_ref, xghi_ref, keep_ref, s16_ref,
              d16_ref, ns_ref, xslo_ref, xshi_ref):
    den = a0_ref[...] + a1_ref[...]
    rden = 1.0 / (den + 1e-16)
    dis = jnp.where(den > 0, lax.rsqrt(jnp.maximum(den, 1e-30)), 0.0)
    ns = asd_ref[...] * keep_ref[...]
    ns = ns + jnp.dot(rden, s16_ref[...], preferred_element_type=jnp.float32)
    ns = ns + jnp.dot(dis, d16_ref[...], preferred_element_type=jnp.float32)
    ns_ref[...] = ns
    dis1 = dis[:, 4:5]
    xslo_ref[...] = xglo_ref[...] * dis1
    xshi_ref[...] = xghi_ref[...] * dis1


def _tc_mid(a0, a1, asd, xglo, xghi, keep, s16, d16):
    grid = (NPAD // BLK,)
    return pl.pallas_call(
        _mid_body,
        grid=grid,
        in_specs=[
            pl.BlockSpec((BLK, 16), lambda i: (i, 0)),
            pl.BlockSpec((BLK, 16), lambda i: (i, 0)),
            pl.BlockSpec((BLK, 16), lambda i: (i, 0)),
            pl.BlockSpec((BLK, 64), lambda i: (i, 0)),
            pl.BlockSpec((BLK, 64), lambda i: (i, 0)),
            pl.BlockSpec((1, 16), lambda i: (0, 0)),
            pl.BlockSpec((16, 16), lambda i: (0, 0)),
            pl.BlockSpec((16, 16), lambda i: (0, 0)),
        ],
        out_specs=[
            pl.BlockSpec((BLK, 16), lambda i: (i, 0)),
            pl.BlockSpec((BLK, 64), lambda i: (i, 0)),
            pl.BlockSpec((BLK, 64), lambda i: (i, 0)),
        ],
        out_shape=[
            jax.ShapeDtypeStruct((NPAD, 16), jnp.float32),
            jax.ShapeDtypeStruct((NPAD, 64), jnp.float32),
            jax.ShapeDtypeStruct((NPAD, 64), jnp.float32),
        ],
    )(a0, a1, asd, xglo, xghi, keep, s16, d16)


def _sc_c_body(src_hbm, dst_hbm, ns_hbm, xwlo_hbm, xwhi_hbm, xglo_hbm, xghi_hbm,
               out_hbm, sidx, didx, ssml, dsml, wbuf, nrm, feats, zbuf,
               acc, sem1, sem2, sem3):
    cid = lax.axis_index("c")
    sid = lax.axis_index("s")
    iota = lax.iota(jnp.int32, 16)
    zero16 = jnp.zeros((16,), jnp.float32)
    rbase = sid * RPT
    ebase0 = sid * EC

    def _zero_acc():
        def _zrow(r, _):
            for v in range(64 // 16):
                zbuf[r, pl.ds(v * 16, 16)] = zero16
            return 0
        lax.fori_loop(0, RPT // 4, _zrow, 0)
        for q in range(4):
            pltpu.sync_copy(zbuf, acc.at[pl.ds(rbase + q * (RPT // 4), RPT // 4)])

    def _gat_half(tab_hbm, hf):
        def _chunk(ch, _):
            base = pl.multiple_of(ebase0 + ch * KC, 8)
            pltpu.sync_copy(src_hbm.at[pl.ds(base, KC)], sidx)
            pltpu.sync_copy(dst_hbm.at[pl.ds(base, KC)], didx)
            cp1 = pltpu.async_copy(ns_hbm.at[sidx], ssml, sem1)
            cp2 = pltpu.async_copy(ns_hbm.at[didx], dsml, sem2)
            cp3 = pltpu.async_copy(tab_hbm.at[sidx], feats, sem3)
            cp1.wait()
            cp2.wait()
            cp3.wait()
            for g in range(KC // 16):
                rows = g * 16 + iota
                for h in (2 * hf, 2 * hf + 1):
                    sa = plsc.load_gather(ssml, [rows, jnp.full((16,), h, jnp.int32)])
                    da = plsc.load_gather(dsml, [rows, jnp.full((16,), 4 + h, jnp.int32)])
                    rd = plsc.load_gather(dsml, [rows, jnp.full((16,), 8 + h, jnp.int32)])
                    al = sa + da
                    al = jnp.where(al > 0, al, 0.2 * al)
                    w = jnp.exp(al) * rd
                    plsc.store_scatter(wbuf, [rows, jnp.full((16,), h, jnp.int32)], w)

            def _edge(j4, _):
                for u in range(4):
                    j = j4 * 4 + u
                    j16 = jnp.full((16,), j, jnp.int32)
                    for hh in range(2):
                        wsp = plsc.load_gather(
                            wbuf, [j16, jnp.full((16,), 2 * hf + hh, jnp.int32)])
                        for half in range(2):
                            v = 2 * hh + half
                            feats[j, pl.ds(v * 16, 16)] = (
                                feats[j, pl.ds(v * 16, 16)] * wsp)
                return 0
            lax.fori_loop(0, KC // 4, _edge, 0)
            pltpu.sync_copy(feats, acc.at[didx], add=True)
            return 0
        lax.fori_loop(0, EC // KC, _chunk, 0)

    def _gcn_half(tab_hbm):
        def _chunk(ch, _):
            base = pl.multiple_of(ebase0 + ch * KC, 8)
            pltpu.sync_copy(src_hbm.at[pl.ds(base, KC)], sidx)
            pltpu.sync_copy(dst_hbm.at[pl.ds(base, KC)], didx)
            pltpu.async_copy(tab_hbm.at[sidx], feats, sem3).wait()
            pltpu.sync_copy(feats, acc.at[didx], add=True)
            return 0
        lax.fori_loop(0, EC // KC, _chunk, 0)

    for hf in range(2):
        _zero_acc()
        plsc.subcore_barrier()

        @pl.when(cid == 0)
        def _gat_core(hf=hf):
            _gat_half(xwlo_hbm if hf == 0 else xwhi_hbm, hf)

        @pl.when(cid == 1)
        def _gcn_core(hf=hf):
            _gcn_half(xglo_hbm if hf == 0 else xghi_hbm)

        plsc.subcore_barrier()
        pltpu.sync_copy(acc.at[pl.ds(rbase, RPT)],
                        out_hbm.at[cid, hf, pl.ds(rbase, RPT)])


def _post_body(gatlo_ref, gathi_ref, gcnlo_ref, gcnhi_ref, ns_ref, x_ref,
               bgat_ref, bgcn_ref, wga_ref, wgb_ref, bgate_ref, gamma_ref,
               beta_ref, o_ref):
    dis1 = ns_ref[:, 12:13]
    gat = jnp.concatenate([gatlo_ref[...], gathi_ref[...]], axis=1) + bgat_ref[...]
    gcn = (jnp.concatenate([gcnlo_ref[...], gcnhi_ref[...]], axis=1) * dis1
           + bgcn_ref[...])
    lg = (jnp.dot(gat, wga_ref[...], preferred_element_type=jnp.float32)
          + jnp.dot(gcn, wgb_ref[...], preferred_element_type=jnp.float32)
          + bgate_ref[...])
    m = jnp.max(lg, axis=-1, keepdims=True)
    eg = jnp.exp(lg - m)
    sm = eg / jnp.sum(eg, axis=-1, keepdims=True)
    out = sm[:, 0:1] * gat + sm[:, 1:2] * gcn
    y = out + x_ref[...]
    mu = jnp.mean(y, axis=-1, keepdims=True)
    yc = y - mu
    var = jnp.mean(yc * yc, axis=-1, keepdims=True)
    o_ref[...] = gamma_ref[...] * yc * lax.rsqrt(var + 1e-5) + beta_ref[...]


def _tc_post(gatlo, gathi, gcnlo, gcnhi, nsmall, xpad, b_gat, b_gcn, wga, wgb,
             b_gate, gamma, beta):
    grid = (NPAD // BLK,)
    return pl.pallas_call(
        _post_body,
        grid=grid,
        in_specs=[
            pl.BlockSpec((BLK, 64), lambda i: (i, 0)),
            pl.BlockSpec((BLK, 64), lambda i: (i, 0)),
            pl.BlockSpec((BLK, 64), lambda i: (i, 0)),
            pl.BlockSpec((BLK, 64), lambda i: (i, 0)),
            pl.BlockSpec((BLK, 16), lambda i: (i, 0)),
            pl.BlockSpec((BLK, D), lambda i: (i, 0)),
            pl.BlockSpec((1, D), lambda i: (0, 0)),
            pl.BlockSpec((1, D), lambda i: (0, 0)),
            pl.BlockSpec((D, 2), lambda i: (0, 0)),
            pl.BlockSpec((D, 2), lambda i: (0, 0)),
            pl.BlockSpec((1, 2), lambda i: (0, 0)),
            pl.BlockSpec((1, D), lambda i: (0, 0)),
            pl.BlockSpec((1, D), lambda i: (0, 0)),
        ],
        out_specs=pl.BlockSpec((BLK, D), lambda i: (i, 0)),
        out_shape=jax.ShapeDtypeStruct((NPAD, D), jnp.float32),
    )(gatlo, gathi, gcnlo, gcnhi, nsmall, xpad, b_gat, b_gcn, wga, wgb,
      b_gate, gamma, beta)


def _sc_kernels():
    if "a" not in _SC_CACHE:
        mesh = plsc.VectorSubcoreMesh(core_axis_name="c", subcore_axis_name="s")
        _SC_CACHE["a"] = pl.kernel(
            _sc_a_body,
            mesh=mesh,
            compiler_params=pltpu.CompilerParams(
                needs_layout_passes=False, use_tc_tiling_on_sc=False),
            out_type=jax.ShapeDtypeStruct((2, NPAD, 16), jnp.float32),
            scratch_types=[
                pltpu.VMEM((KA,), jnp.int32),
                pltpu.VMEM((KA,), jnp.int32),
                pltpu.VMEM((KA, 16), jnp.float32),
                pltpu.VMEM((KA, 16), jnp.float32),
                pltpu.VMEM((KA, 16), jnp.float32),
                pltpu.VMEM((RPT, 16), jnp.float32),
                pltpu.VMEM_SHARED((NPAD, 16), jnp.float32),
                pltpu.SemaphoreType.DMA,
                pltpu.SemaphoreType.DMA,
            ],
        )
        _SC_CACHE["c"] = pl.kernel(
            _sc_c_body,
            mesh=mesh,
            compiler_params=pltpu.CompilerParams(
                needs_layout_passes=False, use_tc_tiling_on_sc=False),
            out_type=jax.ShapeDtypeStruct((2, 2, NPAD, 64), jnp.float32),
            scratch_types=[
                pltpu.VMEM((KC,), jnp.int32),
                pltpu.VMEM((KC,), jnp.int32),
                pltpu.VMEM((KC, 16), jnp.float32),
                pltpu.VMEM((KC, 16), jnp.float32),
                pltpu.VMEM((KC, 16), jnp.float32),
                pltpu.VMEM((KC,), jnp.float32),
                pltpu.VMEM((KC, 64), jnp.float32),
                pltpu.VMEM((RPT // 4, 64), jnp.float32),
                pltpu.VMEM_SHARED((NPAD, 64), jnp.float32),
                pltpu.SemaphoreType.DMA,
                pltpu.SemaphoreType.DMA,
                pltpu.SemaphoreType.DMA,
            ],
        )
    return _SC_CACHE["a"], _SC_CACHE["c"]


def kernel(x, edge_index, W_gat, att_src, att_dst, b_gat, W_gcn, b_gcn,
           W_gate, b_gate, gamma, beta):
    xpad = jnp.pad(x, ((0, NPAD - N), (0, 0)))
    loops = jnp.arange(N, dtype=jnp.int32)
    src = jnp.concatenate([edge_index[0], loops,
                           jnp.zeros((ETPAD - ET,), jnp.int32)])
    dst = jnp.concatenate([edge_index[1], loops,
                           jnp.full((ETPAD - ET,), N, jnp.int32)])

    eyeH = jnp.eye(H, dtype=jnp.float32)
    A_src = (att_src[:, :, None] * eyeH[:, None, :]).reshape(D, H)
    A_dst = (att_dst[:, :, None] * eyeH[:, None, :]).reshape(D, H)
    A_comb = jnp.concatenate([A_src, A_dst, jnp.zeros((D, 8), jnp.float32)], axis=1)

    xwlo, xwhi, xglo, xghi, asd = _tc_pre(xpad, W_gat, W_gcn, A_comb)

    sc_a, sc_c = _sc_kernels()
    accA = sc_a(src, dst, asd)

    col = jnp.arange(16)
    keep = (col < 8).astype(jnp.float32)[None, :]
    s16 = ((col[:, None] + 8 == col[None, :]) & (col[:, None] < 4)).astype(jnp.float32)
    d16 = ((col[:, None] == 4) & (col[None, :] == 12)).astype(jnp.float32)
    nsmall, xslo, xshi = _tc_mid(accA[0], accA[1], asd, xglo, xghi,
                                 keep, s16, d16)

    raw = sc_c(src, dst, nsmall, xwlo, xwhi, xslo, xshi)

    y = _tc_post(raw[0, 0], raw[0, 1], raw[1, 0], raw[1, 1], nsmall, xpad,
                 b_gat[None, :], b_gcn[None, :], W_gate[:D], W_gate[D:],
                 b_gate[None, :], gamma[None, :], beta[None, :])
    return y[:N]


# speedup vs baseline: 39.6615x; 1.2737x over previous
import functools

import jax
import jax.numpy as jnp
from jax import lax
from jax.experimental import pallas as pl
from jax.experimental.pallas import tpu as pltpu
from jax.experimental.pallas import tpu_sc as plsc

N = 10000
D = 128
H = 4
C = 32
E = 320000

NPAD = 10112
RPT = 632
ET = E + N
ETPAD = 330240
EA = ETPAD // 32
EC = ETPAD // 16
KA = 240
KC = 240
BLK = 632

_SC_CACHE = {}


def _pre_body(x_ref, wgat_ref, wgcn_ref, acomb_ref,
              xwlo_ref, xwhi_ref, xglo_ref, xghi_ref, asd_ref):
    x = x_ref[...]
    xw = jnp.dot(x, wgat_ref[...], preferred_element_type=jnp.float32)
    xg = jnp.dot(x, wgcn_ref[...], preferred_element_type=jnp.float32)
    xwlo_ref[...] = xw[:, :64]
    xwhi_ref[...] = xw[:, 64:]
    xglo_ref[...] = xg[:, :64]
    xghi_ref[...] = xg[:, 64:]
    asd_ref[...] = jnp.dot(xw, acomb_ref[...], preferred_element_type=jnp.float32)


def _tc_pre(xpad, W_gat, W_gcn, A_comb):
    grid = (NPAD // BLK,)
    return pl.pallas_call(
        _pre_body,
        grid=grid,
        in_specs=[
            pl.BlockSpec((BLK, D), lambda i: (i, 0)),
            pl.BlockSpec((D, D), lambda i: (0, 0)),
            pl.BlockSpec((D, D), lambda i: (0, 0)),
            pl.BlockSpec((D, 16), lambda i: (0, 0)),
        ],
        out_specs=[
            pl.BlockSpec((BLK, 64), lambda i: (i, 0)),
            pl.BlockSpec((BLK, 64), lambda i: (i, 0)),
            pl.BlockSpec((BLK, 64), lambda i: (i, 0)),
            pl.BlockSpec((BLK, 64), lambda i: (i, 0)),
            pl.BlockSpec((BLK, 16), lambda i: (i, 0)),
        ],
        out_shape=[
            jax.ShapeDtypeStruct((NPAD, 64), jnp.float32),
            jax.ShapeDtypeStruct((NPAD, 64), jnp.float32),
            jax.ShapeDtypeStruct((NPAD, 64), jnp.float32),
            jax.ShapeDtypeStruct((NPAD, 64), jnp.float32),
            jax.ShapeDtypeStruct((NPAD, 16), jnp.float32),
        ],
    )(xpad, W_gat, W_gcn, A_comb)


def _sc_a_body(src_hbm, dst_hbm, asd_hbm, out_hbm,
          sidx, didx, asrc, adst, evec, zbuf, acc, sem1, sem2):
    cid = lax.axis_index("c")
    sid = lax.axis_index("s")
    wid = sid * 2 + cid
    iota = lax.iota(jnp.int32, 16)
    zero16 = jnp.zeros((16,), jnp.float32)
    ecol4 = jnp.where(iota == 4, 1.0, 0.0).astype(jnp.float32)

    def _zrow(r, _):
        zbuf[r, :] = zero16
        return 0
    lax.fori_loop(0, RPT, _zrow, 0)

    def _erow(r, _):
        evec[r, :] = ecol4
        return 0
    lax.fori_loop(0, KA, _erow, 0)

    rbase = sid * RPT
    pltpu.sync_copy(zbuf, acc.at[pl.ds(rbase, RPT)])
    plsc.subcore_barrier()

    def _chunk(ch, _):
        base = pl.multiple_of(wid * EA + ch * KA, 8)
        pltpu.sync_copy(src_hbm.at[pl.ds(base, KA)], sidx)
        pltpu.sync_copy(dst_hbm.at[pl.ds(base, KA)], didx)
        cp1 = pltpu.async_copy(asd_hbm.at[sidx], asrc, sem1)
        cp2 = pltpu.async_copy(asd_hbm.at[didx], adst, sem2)
        cp1.wait()
        cp2.wait()
        for g in range(KA // 16):
            rows = g * 16 + iota
            for h in range(H):
                sa = plsc.load_gather(asrc, [rows, jnp.full((16,), h, jnp.int32)])
                da = plsc.load_gather(adst, [rows, jnp.full((16,), 4 + h, jnp.int32)])
                al = sa + da
                al = jnp.where(al > 0, al, 0.2 * al)
                ev = jnp.exp(al)
                plsc.store_scatter(evec, [rows, jnp.full((16,), h, jnp.int32)], ev)
        pltpu.sync_copy(evec, acc.at[didx], add=True)
        return 0
    lax.fori_loop(0, EA // KA, _chunk, 0)

    plsc.subcore_barrier()
    pltpu.sync_copy(acc.at[pl.ds(rbase, RPT)], out_hbm.at[cid, pl.ds(rbase, RPT)])


def _mid_body(a0_ref, a1_ref, asd_ref, xglo_ref, xghi_ref, keep_ref, s16_ref,
              d16_ref, ns_ref, xslo_ref, xshi_ref):
    den = a0_ref[...] + a1_ref[...]
    rden = 1.0 / (den + 1e-16)
    dis = jnp.where(den > 0, lax.rsqrt(jnp.maximum(den, 1e-30)), 0.0)
    ns = asd_ref[...] * keep_ref[...]
    ns = ns + jnp.dot(rden, s16_ref[...], preferred_element_type=jnp.float32)
    ns = ns + jnp.dot(dis, d16_ref[...], preferred_element_type=jnp.float32)
    ns_ref[...] = ns
    dis1 = dis[:, 4:5]
    xslo_ref[...] = xglo_ref[...] * dis1
    xshi_ref[...] = xghi_ref[...] * dis1


def _tc_mid(a0, a1, asd, xglo, xghi, keep, s16, d16):
    grid = (NPAD // BLK,)
    return pl.pallas_call(
        _mid_body,
        grid=grid,
        in_specs=[
            pl.BlockSpec((BLK, 16), lambda i: (i, 0)),
            pl.BlockSpec((BLK, 16), lambda i: (i, 0)),
            pl.BlockSpec((BLK, 16), lambda i: (i, 0)),
            pl.BlockSpec((BLK, 64), lambda i: (i, 0)),
            pl.BlockSpec((BLK, 64), lambda i: (i, 0)),
            pl.BlockSpec((1, 16), lambda i: (0, 0)),
            pl.BlockSpec((16, 16), lambda i: (0, 0)),
            pl.BlockSpec((16, 16), lambda i: (0, 0)),
        ],
        out_specs=[
            pl.BlockSpec((BLK, 16), lambda i: (i, 0)),
            pl.BlockSpec((BLK, 64), lambda i: (i, 0)),
            pl.BlockSpec((BLK, 64), lambda i: (i, 0)),
        ],
        out_shape=[
            jax.ShapeDtypeStruct((NPAD, 16), jnp.float32),
            jax.ShapeDtypeStruct((NPAD, 64), jnp.float32),
            jax.ShapeDtypeStruct((NPAD, 64), jnp.float32),
        ],
    )(a0, a1, asd, xglo, xghi, keep, s16, d16)


def _sc_c_body(src_hbm, dst_hbm, ns_hbm, xwlo_hbm, xwhi_hbm, xglo_hbm, xghi_hbm,
               out_hbm, sidx, didx, ssml, dsml, wbuf, nrm, feats, zbuf,
               acc, sem1, sem2, sem3):
    cid = lax.axis_index("c")
    sid = lax.axis_index("s")
    iota = lax.iota(jnp.int32, 16)
    zero16 = jnp.zeros((16,), jnp.float32)
    rbase = sid * RPT
    ebase0 = sid * EC

    def _zero_acc():
        def _zrow(r, _):
            for v in range(64 // 16):
                zbuf[r, pl.ds(v * 16, 16)] = zero16
            return 0
        lax.fori_loop(0, RPT // 4, _zrow, 0)
        for q in range(4):
            pltpu.sync_copy(zbuf, acc.at[pl.ds(rbase + q * (RPT // 4), RPT // 4)])

    def _gat_half(tab_hbm, hf):
        def _chunk(ch, _):
            base = pl.multiple_of(ebase0 + ch * KC, 8)
            pltpu.sync_copy(src_hbm.at[pl.ds(base, KC)], sidx)
            pltpu.sync_copy(dst_hbm.at[pl.ds(base, KC)], didx)
            cp1 = pltpu.async_copy(ns_hbm.at[sidx], ssml, sem1)
            cp2 = pltpu.async_copy(ns_hbm.at[didx], dsml, sem2)
            cp3 = pltpu.async_copy(tab_hbm.at[sidx], feats, sem3)
            cp1.wait()
            cp2.wait()
            cp3.wait()
            for g in range(KC // 16):
                rows = g * 16 + iota
                for h in (2 * hf, 2 * hf + 1):
                    sa = plsc.load_gather(ssml, [rows, jnp.full((16,), h, jnp.int32)])
                    da = plsc.load_gather(dsml, [rows, jnp.full((16,), 4 + h, jnp.int32)])
                    rd = plsc.load_gather(dsml, [rows, jnp.full((16,), 8 + h, jnp.int32)])
                    al = sa + da
                    al = jnp.where(al > 0, al, 0.2 * al)
                    w = jnp.exp(al) * rd
                    plsc.store_scatter(wbuf, [rows, jnp.full((16,), h, jnp.int32)], w)

            def _edge(j4, _):
                for u in range(4):
                    j = j4 * 4 + u
                    j16 = jnp.full((16,), j, jnp.int32)
                    for hh in range(2):
                        wsp = plsc.load_gather(
                            wbuf, [j16, jnp.full((16,), 2 * hf + hh, jnp.int32)])
                        for half in range(2):
                            v = 2 * hh + half
                            feats[j, pl.ds(v * 16, 16)] = (
                                feats[j, pl.ds(v * 16, 16)] * wsp)
                return 0
            lax.fori_loop(0, KC // 4, _edge, 0)
            pltpu.sync_copy(feats, acc.at[didx], add=True)
            return 0
        lax.fori_loop(0, EC // KC, _chunk, 0)

    def _gcn_half(tab_hbm):
        def _chunk(ch, _):
            base = pl.multiple_of(ebase0 + ch * KC, 8)
            pltpu.sync_copy(src_hbm.at[pl.ds(base, KC)], sidx)
            pltpu.sync_copy(dst_hbm.at[pl.ds(base, KC)], didx)
            pltpu.async_copy(tab_hbm.at[sidx], feats, sem3).wait()
            pltpu.sync_copy(feats, acc.at[didx], add=True)
            return 0
        lax.fori_loop(0, EC // KC, _chunk, 0)

    for p in range(2):
        _zero_acc()
        plsc.subcore_barrier()

        if p == 0:
            @pl.when(cid == 0)
            def _gat_lo():
                _gat_half(xwlo_hbm, 0)

            @pl.when(cid == 1)
            def _gat_hi():
                _gat_half(xwhi_hbm, 1)
        else:
            @pl.when(cid == 0)
            def _gcn_lo():
                _gcn_half(xglo_hbm)

            @pl.when(cid == 1)
            def _gcn_hi():
                _gcn_half(xghi_hbm)

        plsc.subcore_barrier()
        pltpu.sync_copy(acc.at[pl.ds(rbase, RPT)],
                        out_hbm.at[p, cid, pl.ds(rbase, RPT)])


def _post_body(gatlo_ref, gathi_ref, gcnlo_ref, gcnhi_ref, ns_ref, x_ref,
               bgat_ref, bgcn_ref, wga_ref, wgb_ref, bgate_ref, gamma_ref,
               beta_ref, o_ref):
    dis1 = ns_ref[:, 12:13]
    gat = jnp.concatenate([gatlo_ref[...], gathi_ref[...]], axis=1) + bgat_ref[...]
    gcn = (jnp.concatenate([gcnlo_ref[...], gcnhi_ref[...]], axis=1) * dis1
           + bgcn_ref[...])
    lg = (jnp.dot(gat, wga_ref[...], preferred_element_type=jnp.float32)
          + jnp.dot(gcn, wgb_ref[...], preferred_element_type=jnp.float32)
          + bgate_ref[...])
    m = jnp.max(lg, axis=-1, keepdims=True)
    eg = jnp.exp(lg - m)
    sm = eg / jnp.sum(eg, axis=-1, keepdims=True)
    out = sm[:, 0:1] * gat + sm[:, 1:2] * gcn
    y = out + x_ref[...]
    mu = jnp.mean(y, axis=-1, keepdims=True)
    yc = y - mu
    var = jnp.mean(yc * yc, axis=-1, keepdims=True)
    o_ref[...] = gamma_ref[...] * yc * lax.rsqrt(var + 1e-5) + beta_ref[...]


def _tc_post(gatlo, gathi, gcnlo, gcnhi, nsmall, xpad, b_gat, b_gcn, wga, wgb,
             b_gate, gamma, beta):
    grid = (NPAD // BLK,)
    return pl.pallas_call(
        _post_body,
        grid=grid,
        in_specs=[
            pl.BlockSpec((BLK, 64), lambda i: (i, 0)),
            pl.BlockSpec((BLK, 64), lambda i: (i, 0)),
            pl.BlockSpec((BLK, 64), lambda i: (i, 0)),
            pl.BlockSpec((BLK, 64), lambda i: (i, 0)),
            pl.BlockSpec((BLK, 16), lambda i: (i, 0)),
            pl.BlockSpec((BLK, D), lambda i: (i, 0)),
            pl.BlockSpec((1, D), lambda i: (0, 0)),
            pl.BlockSpec((1, D), lambda i: (0, 0)),
            pl.BlockSpec((D, 2), lambda i: (0, 0)),
            pl.BlockSpec((D, 2), lambda i: (0, 0)),
            pl.BlockSpec((1, 2), lambda i: (0, 0)),
            pl.BlockSpec((1, D), lambda i: (0, 0)),
            pl.BlockSpec((1, D), lambda i: (0, 0)),
        ],
        out_specs=pl.BlockSpec((BLK, D), lambda i: (i, 0)),
        out_shape=jax.ShapeDtypeStruct((NPAD, D), jnp.float32),
    )(gatlo, gathi, gcnlo, gcnhi, nsmall, xpad, b_gat, b_gcn, wga, wgb,
      b_gate, gamma, beta)


def _sc_kernels():
    if "a" not in _SC_CACHE:
        mesh = plsc.VectorSubcoreMesh(core_axis_name="c", subcore_axis_name="s")
        _SC_CACHE["a"] = pl.kernel(
            _sc_a_body,
            mesh=mesh,
            compiler_params=pltpu.CompilerParams(
                needs_layout_passes=False, use_tc_tiling_on_sc=False),
            out_type=jax.ShapeDtypeStruct((2, NPAD, 16), jnp.float32),
            scratch_types=[
                pltpu.VMEM((KA,), jnp.int32),
                pltpu.VMEM((KA,), jnp.int32),
                pltpu.VMEM((KA, 16), jnp.float32),
                pltpu.VMEM((KA, 16), jnp.float32),
                pltpu.VMEM((KA, 16), jnp.float32),
                pltpu.VMEM((RPT, 16), jnp.float32),
                pltpu.VMEM_SHARED((NPAD, 16), jnp.float32),
                pltpu.SemaphoreType.DMA,
                pltpu.SemaphoreType.DMA,
            ],
        )
        _SC_CACHE["c"] = pl.kernel(
            _sc_c_body,
            mesh=mesh,
            compiler_params=pltpu.CompilerParams(
                needs_layout_passes=False, use_tc_tiling_on_sc=False),
            out_type=jax.ShapeDtypeStruct((2, 2, NPAD, 64), jnp.float32),
            scratch_types=[
                pltpu.VMEM((KC,), jnp.int32),
                pltpu.VMEM((KC,), jnp.int32),
                pltpu.VMEM((KC, 16), jnp.float32),
                pltpu.VMEM((KC, 16), jnp.float32),
                pltpu.VMEM((KC, 16), jnp.float32),
                pltpu.VMEM((KC,), jnp.float32),
                pltpu.VMEM((KC, 64), jnp.float32),
                pltpu.VMEM((RPT // 4, 64), jnp.float32),
                pltpu.VMEM_SHARED((NPAD, 64), jnp.float32),
                pltpu.SemaphoreType.DMA,
                pltpu.SemaphoreType.DMA,
                pltpu.SemaphoreType.DMA,
            ],
        )
    return _SC_CACHE["a"], _SC_CACHE["c"]


def kernel(x, edge_index, W_gat, att_src, att_dst, b_gat, W_gcn, b_gcn,
           W_gate, b_gate, gamma, beta):
    xpad = jnp.pad(x, ((0, NPAD - N), (0, 0)))
    loops = jnp.arange(N, dtype=jnp.int32)
    src = jnp.concatenate([edge_index[0], loops,
                           jnp.zeros((ETPAD - ET,), jnp.int32)])
    dst = jnp.concatenate([edge_index[1], loops,
                           jnp.full((ETPAD - ET,), N, jnp.int32)])

    eyeH = jnp.eye(H, dtype=jnp.float32)
    A_src = (att_src[:, :, None] * eyeH[:, None, :]).reshape(D, H)
    A_dst = (att_dst[:, :, None] * eyeH[:, None, :]).reshape(D, H)
    A_comb = jnp.concatenate([A_src, A_dst, jnp.zeros((D, 8), jnp.float32)], axis=1)

    xwlo, xwhi, xglo, xghi, asd = _tc_pre(xpad, W_gat, W_gcn, A_comb)

    sc_a, sc_c = _sc_kernels()
    accA = sc_a(src, dst, asd)

    col = jnp.arange(16)
    keep = (col < 8).astype(jnp.float32)[None, :]
    s16 = ((col[:, None] + 8 == col[None, :]) & (col[:, None] < 4)).astype(jnp.float32)
    d16 = ((col[:, None] == 4) & (col[None, :] == 12)).astype(jnp.float32)
    nsmall, xslo, xshi = _tc_mid(accA[0], accA[1], asd, xglo, xghi,
                                 keep, s16, d16)

    raw = sc_c(src, dst, nsmall, xwlo, xwhi, xslo, xshi)

    y = _tc_post(raw[0, 0], raw[0, 1], raw[1, 0], raw[1, 1], nsmall, xpad,
                 b_gat[None, :], b_gcn[None, :], W_gate[:D], W_gate[D:],
                 b_gate[None, :], gamma[None, :], beta[None, :])
    return y[:N]


# speedup vs baseline: 46.5065x; 1.1726x over previous
import functools

import jax
import jax.numpy as jnp
from jax import lax
from jax.experimental import pallas as pl
from jax.experimental.pallas import tpu as pltpu
from jax.experimental.pallas import tpu_sc as plsc

N = 10000
D = 128
H = 4
C = 32
E = 320000

NPAD = 10112
RPT = 632
ET = E + N
ETPAD = 330240
EA = ETPAD // 32
EC = ETPAD // 16
KA = 240
KC = 240
BLK = 632

_SC_CACHE = {}


def _pre_body(x_ref, wgat_ref, wgcn_ref, acomb_ref,
              xwlo_ref, xwhi_ref, xglo_ref, xghi_ref, asd_ref):
    x = x_ref[...]
    xw = jnp.dot(x, wgat_ref[...], preferred_element_type=jnp.float32)
    xg = jnp.dot(x, wgcn_ref[...], preferred_element_type=jnp.float32)
    xwlo_ref[...] = xw[:, :64]
    xwhi_ref[...] = xw[:, 64:]
    xglo_ref[...] = xg[:, :64]
    xghi_ref[...] = xg[:, 64:]
    asd_ref[...] = jnp.dot(xw, acomb_ref[...], preferred_element_type=jnp.float32)


def _tc_pre(xpad, W_gat, W_gcn, A_comb):
    grid = (NPAD // BLK,)
    return pl.pallas_call(
        _pre_body,
        grid=grid,
        in_specs=[
            pl.BlockSpec((BLK, D), lambda i: (i, 0)),
            pl.BlockSpec((D, D), lambda i: (0, 0)),
            pl.BlockSpec((D, D), lambda i: (0, 0)),
            pl.BlockSpec((D, 16), lambda i: (0, 0)),
        ],
        out_specs=[
            pl.BlockSpec((BLK, 64), lambda i: (i, 0)),
            pl.BlockSpec((BLK, 64), lambda i: (i, 0)),
            pl.BlockSpec((BLK, 64), lambda i: (i, 0)),
            pl.BlockSpec((BLK, 64), lambda i: (i, 0)),
            pl.BlockSpec((BLK, 16), lambda i: (i, 0)),
        ],
        out_shape=[
            jax.ShapeDtypeStruct((NPAD, 64), jnp.float32),
            jax.ShapeDtypeStruct((NPAD, 64), jnp.float32),
            jax.ShapeDtypeStruct((NPAD, 64), jnp.float32),
            jax.ShapeDtypeStruct((NPAD, 64), jnp.float32),
            jax.ShapeDtypeStruct((NPAD, 16), jnp.float32),
        ],
    )(xpad, W_gat, W_gcn, A_comb)


def _sc_a_body(src_hbm, dst_hbm, asd_hbm, out_hbm,
          sidx, didx, asrc, adst, evec, zbuf, acc, sem1, sem2):
    cid = lax.axis_index("c")
    sid = lax.axis_index("s")
    wid = sid * 2 + cid
    iota = lax.iota(jnp.int32, 16)
    zero16 = jnp.zeros((16,), jnp.float32)
    ecol4 = jnp.where(iota == 4, 1.0, 0.0).astype(jnp.float32)

    def _zrow(r, _):
        zbuf[r, :] = zero16
        return 0
    lax.fori_loop(0, RPT, _zrow, 0)

    def _erow(r, _):
        evec[r, :] = ecol4
        return 0
    lax.fori_loop(0, KA, _erow, 0)

    rbase = sid * RPT
    pltpu.sync_copy(zbuf, acc.at[pl.ds(rbase, RPT)])
    plsc.subcore_barrier()

    def _chunk(ch, _):
        base = pl.multiple_of(wid * EA + ch * KA, 8)
        pltpu.sync_copy(src_hbm.at[pl.ds(base, KA)], sidx)
        pltpu.sync_copy(dst_hbm.at[pl.ds(base, KA)], didx)
        cp1 = pltpu.async_copy(asd_hbm.at[sidx], asrc, sem1)
        cp2 = pltpu.async_copy(asd_hbm.at[didx], adst, sem2)
        cp1.wait()
        cp2.wait()
        for g in range(KA // 16):
            rows = g * 16 + iota
            for h in range(H):
                sa = plsc.load_gather(asrc, [rows, jnp.full((16,), h, jnp.int32)])
                da = plsc.load_gather(adst, [rows, jnp.full((16,), 4 + h, jnp.int32)])
                al = sa + da
                al = jnp.where(al > 0, al, 0.2 * al)
                ev = jnp.exp(al)
                plsc.store_scatter(evec, [rows, jnp.full((16,), h, jnp.int32)], ev)
        pltpu.sync_copy(evec, acc.at[didx], add=True)
        return 0
    lax.fori_loop(0, EA // KA, _chunk, 0)

    plsc.subcore_barrier()
    pltpu.sync_copy(acc.at[pl.ds(rbase, RPT)], out_hbm.at[cid, pl.ds(rbase, RPT)])


def _mid_body(a0_ref, a1_ref, asd_ref, xglo_ref, xghi_ref, keep_ref, s16_ref,
              d16_ref, ns_ref, xslo_ref, xshi_ref):
    den = a0_ref[...] + a1_ref[...]
    rden = 1.0 / (den + 1e-16)
    dis = jnp.where(den > 0, lax.rsqrt(jnp.maximum(den, 1e-30)), 0.0)
    ns = asd_ref[...] * keep_ref[...]
    ns = ns + jnp.dot(rden, s16_ref[...], preferred_element_type=jnp.float32)
    ns = ns + jnp.dot(dis, d16_ref[...], preferred_element_type=jnp.float32)
    ns_ref[...] = ns
    dis1 = dis[:, 4:5]
    xslo_ref[...] = xglo_ref[...] * dis1
    xshi_ref[...] = xghi_ref[...] * dis1


def _tc_mid(a0, a1, asd, xglo, xghi, keep, s16, d16):
    grid = (NPAD // BLK,)
    return pl.pallas_call(
        _mid_body,
        grid=grid,
        in_specs=[
            pl.BlockSpec((BLK, 16), lambda i: (i, 0)),
            pl.BlockSpec((BLK, 16), lambda i: (i, 0)),
            pl.BlockSpec((BLK, 16), lambda i: (i, 0)),
            pl.BlockSpec((BLK, 64), lambda i: (i, 0)),
            pl.BlockSpec((BLK, 64), lambda i: (i, 0)),
            pl.BlockSpec((1, 16), lambda i: (0, 0)),
            pl.BlockSpec((16, 16), lambda i: (0, 0)),
            pl.BlockSpec((16, 16), lambda i: (0, 0)),
        ],
        out_specs=[
            pl.BlockSpec((BLK, 16), lambda i: (i, 0)),
            pl.BlockSpec((BLK, 64), lambda i: (i, 0)),
            pl.BlockSpec((BLK, 64), lambda i: (i, 0)),
        ],
        out_shape=[
            jax.ShapeDtypeStruct((NPAD, 16), jnp.float32),
            jax.ShapeDtypeStruct((NPAD, 64), jnp.float32),
            jax.ShapeDtypeStruct((NPAD, 64), jnp.float32),
        ],
    )(a0, a1, asd, xglo, xghi, keep, s16, d16)


def _sc_c_body(src_hbm, dst_hbm, ns_hbm, xwlo_hbm, xwhi_hbm, xglo_hbm, xghi_hbm,
               out_hbm, sidx, didx, ssml, dsml, wbuf, feats, zbuf, acc,
               gsem0, gsem1, ssem0, ssem1, dsem0, dsem1, csem0, csem1):
    gsem = (gsem0, gsem1)
    ssem = (ssem0, ssem1)
    dsem = (dsem0, dsem1)
    csem = (csem0, csem1)
    cid = lax.axis_index("c")
    sid = lax.axis_index("s")
    iota = lax.iota(jnp.int32, 16)
    zero16 = jnp.zeros((16,), jnp.float32)
    rbase = sid * RPT
    ebase0 = sid * EC

    def _zero_acc():
        def _zrow(r, _):
            for v in range(64 // 16):
                zbuf[r, pl.ds(v * 16, 16)] = zero16
            return 0
        lax.fori_loop(0, RPT // 4, _zrow, 0)
        for q in range(4):
            pltpu.sync_copy(zbuf, acc.at[pl.ds(rbase + q * (RPT // 4), RPT // 4)])

    NCH = EC // KC

    def _load_idx(ch, b):
        base = pl.multiple_of(ebase0 + ch * KC, 8)
        pltpu.sync_copy(src_hbm.at[pl.ds(base, KC)], sidx.at[b])
        pltpu.sync_copy(dst_hbm.at[pl.ds(base, KC)], didx.at[b])

    def _pipe(tab_hbm, gat_hf):
        is_gat = gat_hf is not None

        def _issue(b):
            pltpu.async_copy(tab_hbm.at[sidx.at[b]], feats.at[b], gsem[b])
            if is_gat:
                pltpu.async_copy(ns_hbm.at[sidx.at[b]], ssml.at[b], ssem[b])
                pltpu.async_copy(ns_hbm.at[didx.at[b]], dsml.at[b], dsem[b])

        def _wait_gather(b):
            pltpu.make_async_copy(tab_hbm.at[sidx.at[b]], feats.at[b],
                                  gsem[b]).wait()
            if is_gat:
                pltpu.make_async_copy(ns_hbm.at[sidx.at[b]], ssml.at[b],
                                      ssem[b]).wait()
                pltpu.make_async_copy(ns_hbm.at[didx.at[b]], dsml.at[b],
                                      dsem[b]).wait()

        def _wait_scatter(b):
            pltpu.make_async_copy(feats.at[b], acc.at[didx.at[b]],
                                  csem[b]).wait()

        _load_idx(0, 0)
        _issue(0)

        def _pair(gg, _):
            for b in (0, 1):
                ch = gg * 2 + b
                nb = 1 - b
                _wait_gather(b)

                @pl.when(ch + 1 < NCH)
                def _prefetch():
                    @pl.when(ch >= 1)
                    def _drain():
                        _wait_scatter(nb)
                    _load_idx(ch + 1, nb)
                    _issue(nb)

                if is_gat:
                    bb = jnp.full((16,), b, jnp.int32)
                    for g in range(KC // 16):
                        rows = g * 16 + iota
                        for h in (2 * gat_hf, 2 * gat_hf + 1):
                            sa = plsc.load_gather(
                                ssml, [bb, rows, jnp.full((16,), h, jnp.int32)])
                            da = plsc.load_gather(
                                dsml, [bb, rows, jnp.full((16,), 4 + h, jnp.int32)])
                            rd = plsc.load_gather(
                                dsml, [bb, rows, jnp.full((16,), 8 + h, jnp.int32)])
                            al = sa + da
                            al = jnp.where(al > 0, al, 0.2 * al)
                            w = jnp.exp(al) * rd
                            plsc.store_scatter(
                                wbuf, [rows, jnp.full((16,), h, jnp.int32)], w)

                    def _edge(j4, _):
                        for u in range(4):
                            j = j4 * 4 + u
                            j16 = jnp.full((16,), j, jnp.int32)
                            for hh in range(2):
                                wsp = plsc.load_gather(
                                    wbuf,
                                    [j16, jnp.full((16,), 2 * gat_hf + hh,
                                                   jnp.int32)])
                                for half in range(2):
                                    v = 2 * hh + half
                                    feats[b, j, pl.ds(v * 16, 16)] = (
                                        feats[b, j, pl.ds(v * 16, 16)] * wsp)
                        return 0
                    lax.fori_loop(0, KC // 4, _edge, 0)

                pltpu.async_copy(feats.at[b], acc.at[didx.at[b]], csem[b],
                                 add=True)
            return 0
        lax.fori_loop(0, NCH // 2, _pair, 0)
        _wait_scatter(0)
        _wait_scatter(1)

    for p in range(2):
        _zero_acc()
        plsc.subcore_barrier()

        if p == 0:
            @pl.when(cid == 0)
            def _gat_lo():
                _pipe(xwlo_hbm, 0)

            @pl.when(cid == 1)
            def _gat_hi():
                _pipe(xwhi_hbm, 1)
        else:
            @pl.when(cid == 0)
            def _gcn_lo():
                _pipe(xglo_hbm, None)

            @pl.when(cid == 1)
            def _gcn_hi():
                _pipe(xghi_hbm, None)

        plsc.subcore_barrier()
        pltpu.sync_copy(acc.at[pl.ds(rbase, RPT)],
                        out_hbm.at[p, cid, pl.ds(rbase, RPT)])


def _post_body(gatlo_ref, gathi_ref, gcnlo_ref, gcnhi_ref, ns_ref, x_ref,
               bgat_ref, bgcn_ref, wga_ref, wgb_ref, bgate_ref, gamma_ref,
               beta_ref, o_ref):
    dis1 = ns_ref[:, 12:13]
    gat = jnp.concatenate([gatlo_ref[...], gathi_ref[...]], axis=1) + bgat_ref[...]
    gcn = (jnp.concatenate([gcnlo_ref[...], gcnhi_ref[...]], axis=1) * dis1
           + bgcn_ref[...])
    lg = (jnp.dot(gat, wga_ref[...], preferred_element_type=jnp.float32)
          + jnp.dot(gcn, wgb_ref[...], preferred_element_type=jnp.float32)
          + bgate_ref[...])
    m = jnp.max(lg, axis=-1, keepdims=True)
    eg = jnp.exp(lg - m)
    sm = eg / jnp.sum(eg, axis=-1, keepdims=True)
    out = sm[:, 0:1] * gat + sm[:, 1:2] * gcn
    y = out + x_ref[...]
    mu = jnp.mean(y, axis=-1, keepdims=True)
    yc = y - mu
    var = jnp.mean(yc * yc, axis=-1, keepdims=True)
    o_ref[...] = gamma_ref[...] * yc * lax.rsqrt(var + 1e-5) + beta_ref[...]


def _tc_post(gatlo, gathi, gcnlo, gcnhi, nsmall, xpad, b_gat, b_gcn, wga, wgb,
             b_gate, gamma, beta):
    grid = (NPAD // BLK,)
    return pl.pallas_call(
        _post_body,
        grid=grid,
        in_specs=[
            pl.BlockSpec((BLK, 64), lambda i: (i, 0)),
            pl.BlockSpec((BLK, 64), lambda i: (i, 0)),
            pl.BlockSpec((BLK, 64), lambda i: (i, 0)),
            pl.BlockSpec((BLK, 64), lambda i: (i, 0)),
            pl.BlockSpec((BLK, 16), lambda i: (i, 0)),
            pl.BlockSpec((BLK, D), lambda i: (i, 0)),
            pl.BlockSpec((1, D), lambda i: (0, 0)),
            pl.BlockSpec((1, D), lambda i: (0, 0)),
            pl.BlockSpec((D, 2), lambda i: (0, 0)),
            pl.BlockSpec((D, 2), lambda i: (0, 0)),
            pl.BlockSpec((1, 2), lambda i: (0, 0)),
            pl.BlockSpec((1, D), lambda i: (0, 0)),
            pl.BlockSpec((1, D), lambda i: (0, 0)),
        ],
        out_specs=pl.BlockSpec((BLK, D), lambda i: (i, 0)),
        out_shape=jax.ShapeDtypeStruct((NPAD, D), jnp.float32),
    )(gatlo, gathi, gcnlo, gcnhi, nsmall, xpad, b_gat, b_gcn, wga, wgb,
      b_gate, gamma, beta)


def _sc_kernels():
    if "a" not in _SC_CACHE:
        mesh = plsc.VectorSubcoreMesh(core_axis_name="c", subcore_axis_name="s")
        _SC_CACHE["a"] = pl.kernel(
            _sc_a_body,
            mesh=mesh,
            compiler_params=pltpu.CompilerParams(
                needs_layout_passes=False, use_tc_tiling_on_sc=False),
            out_type=jax.ShapeDtypeStruct((2, NPAD, 16), jnp.float32),
            scratch_types=[
                pltpu.VMEM((KA,), jnp.int32),
                pltpu.VMEM((KA,), jnp.int32),
                pltpu.VMEM((KA, 16), jnp.float32),
                pltpu.VMEM((KA, 16), jnp.float32),
                pltpu.VMEM((KA, 16), jnp.float32),
                pltpu.VMEM((RPT, 16), jnp.float32),
                pltpu.VMEM_SHARED((NPAD, 16), jnp.float32),
                pltpu.SemaphoreType.DMA,
                pltpu.SemaphoreType.DMA,
            ],
        )
        _SC_CACHE["c"] = pl.kernel(
            _sc_c_body,
            mesh=mesh,
            compiler_params=pltpu.CompilerParams(
                needs_layout_passes=False, use_tc_tiling_on_sc=False),
            out_type=jax.ShapeDtypeStruct((2, 2, NPAD, 64), jnp.float32),
            scratch_types=[
                pltpu.VMEM((2, KC), jnp.int32),
                pltpu.VMEM((2, KC), jnp.int32),
                pltpu.VMEM((2, KC, 16), jnp.float32),
                pltpu.VMEM((2, KC, 16), jnp.float32),
                pltpu.VMEM((KC, 16), jnp.float32),
                pltpu.VMEM((2, KC, 64), jnp.float32),
                pltpu.VMEM((RPT // 4, 64), jnp.float32),
                pltpu.VMEM_SHARED((NPAD, 64), jnp.float32),
                pltpu.SemaphoreType.DMA,
                pltpu.SemaphoreType.DMA,
                pltpu.SemaphoreType.DMA,
                pltpu.SemaphoreType.DMA,
                pltpu.SemaphoreType.DMA,
                pltpu.SemaphoreType.DMA,
                pltpu.SemaphoreType.DMA,
                pltpu.SemaphoreType.DMA,
            ],
        )
    return _SC_CACHE["a"], _SC_CACHE["c"]


def kernel(x, edge_index, W_gat, att_src, att_dst, b_gat, W_gcn, b_gcn,
           W_gate, b_gate, gamma, beta):
    xpad = jnp.pad(x, ((0, NPAD - N), (0, 0)))
    loops = jnp.arange(N, dtype=jnp.int32)
    src = jnp.concatenate([edge_index[0], loops,
                           jnp.zeros((ETPAD - ET,), jnp.int32)])
    dst = jnp.concatenate([edge_index[1], loops,
                           jnp.full((ETPAD - ET,), N, jnp.int32)])

    eyeH = jnp.eye(H, dtype=jnp.float32)
    A_src = (att_src[:, :, None] * eyeH[:, None, :]).reshape(D, H)
    A_dst = (att_dst[:, :, None] * eyeH[:, None, :]).reshape(D, H)
    A_comb = jnp.concatenate([A_src, A_dst, jnp.zeros((D, 8), jnp.float32)], axis=1)

    xwlo, xwhi, xglo, xghi, asd = _tc_pre(xpad, W_gat, W_gcn, A_comb)

    sc_a, sc_c = _sc_kernels()
    accA = sc_a(src, dst, asd)

    col = jnp.arange(16)
    keep = (col < 8).astype(jnp.float32)[None, :]
    s16 = ((col[:, None] + 8 == col[None, :]) & (col[:, None] < 4)).astype(jnp.float32)
    d16 = ((col[:, None] == 4) & (col[None, :] == 12)).astype(jnp.float32)
    nsmall, xslo, xshi = _tc_mid(accA[0], accA[1], asd, xglo, xghi,
                                 keep, s16, d16)

    raw = sc_c(src, dst, nsmall, xwlo, xwhi, xslo, xshi)

    y = _tc_post(raw[0, 0], raw[0, 1], raw[1, 0], raw[1, 1], nsmall, xpad,
                 b_gat[None, :], b_gcn[None, :], W_gate[:D], W_gate[D:],
                 b_gate[None, :], gamma[None, :], beta[None, :])
    return y[:N]


# speedup vs baseline: 48.2740x; 1.0380x over previous
import functools

import jax
import jax.numpy as jnp
from jax import lax
from jax.experimental import pallas as pl
from jax.experimental.pallas import tpu as pltpu
from jax.experimental.pallas import tpu_sc as plsc

N = 10000
D = 128
H = 4
C = 32
E = 320000

NPAD = 10112
RPT = 632
ET = E + N
ETPAD = 330240
EA = ETPAD // 32
EC = ETPAD // 16
KA = 240
KC = 240
BLK = 632

_SC_CACHE = {}


def _pre_body(x_ref, wgat_ref, wgcn_ref, acomb_ref,
              xwlo_ref, xwhi_ref, xglo_ref, xghi_ref, asd_ref):
    x = x_ref[...]
    xw = jnp.dot(x, wgat_ref[...], preferred_element_type=jnp.float32)
    xg = jnp.dot(x, wgcn_ref[...], preferred_element_type=jnp.float32)
    xwlo_ref[...] = xw[:, :64]
    xwhi_ref[...] = xw[:, 64:]
    xglo_ref[...] = xg[:, :64]
    xghi_ref[...] = xg[:, 64:]
    asd_ref[...] = jnp.dot(xw, acomb_ref[...], preferred_element_type=jnp.float32)


def _tc_pre(xpad, W_gat, W_gcn, A_comb):
    grid = (NPAD // BLK,)
    return pl.pallas_call(
        _pre_body,
        grid=grid,
        in_specs=[
            pl.BlockSpec((BLK, D), lambda i: (i, 0)),
            pl.BlockSpec((D, D), lambda i: (0, 0)),
            pl.BlockSpec((D, D), lambda i: (0, 0)),
            pl.BlockSpec((D, 16), lambda i: (0, 0)),
        ],
        out_specs=[
            pl.BlockSpec((BLK, 64), lambda i: (i, 0)),
            pl.BlockSpec((BLK, 64), lambda i: (i, 0)),
            pl.BlockSpec((BLK, 64), lambda i: (i, 0)),
            pl.BlockSpec((BLK, 64), lambda i: (i, 0)),
            pl.BlockSpec((BLK, 16), lambda i: (i, 0)),
        ],
        out_shape=[
            jax.ShapeDtypeStruct((NPAD, 64), jnp.float32),
            jax.ShapeDtypeStruct((NPAD, 64), jnp.float32),
            jax.ShapeDtypeStruct((NPAD, 64), jnp.float32),
            jax.ShapeDtypeStruct((NPAD, 64), jnp.float32),
            jax.ShapeDtypeStruct((NPAD, 16), jnp.float32),
        ],
    )(xpad, W_gat, W_gcn, A_comb)


def _sc_a_body(src_hbm, dst_hbm, asd_hbm, out_hbm,
          sidx, didx, asrc, adst, evec, zbuf, acc,
          gsem0, gsem1, ssem0, ssem1, csem0, csem1):
    gsem = (gsem0, gsem1)
    ssem = (ssem0, ssem1)
    csem = (csem0, csem1)
    cid = lax.axis_index("c")
    sid = lax.axis_index("s")
    wid = sid * 2 + cid
    iota = lax.iota(jnp.int32, 16)
    zero16 = jnp.zeros((16,), jnp.float32)
    ecol4 = jnp.where(iota == 4, 1.0, 0.0).astype(jnp.float32)
    NCH = EA // KA

    def _zrow(r, _):
        zbuf[r, :] = zero16
        return 0
    lax.fori_loop(0, RPT, _zrow, 0)

    def _erow(r, _):
        for b in range(2):
            evec[b, r, :] = ecol4
        return 0
    lax.fori_loop(0, KA, _erow, 0)

    rbase = sid * RPT
    pltpu.sync_copy(zbuf, acc.at[pl.ds(rbase, RPT)])
    plsc.subcore_barrier()

    def _load_idx(ch, b):
        base = pl.multiple_of(wid * EA + ch * KA, 8)
        pltpu.sync_copy(src_hbm.at[pl.ds(base, KA)], sidx.at[b])
        pltpu.sync_copy(dst_hbm.at[pl.ds(base, KA)], didx.at[b])

    def _issue(b):
        pltpu.async_copy(asd_hbm.at[sidx.at[b]], asrc.at[b], gsem[b])
        pltpu.async_copy(asd_hbm.at[didx.at[b]], adst.at[b], ssem[b])

    def _wait_gather(b):
        pltpu.make_async_copy(asd_hbm.at[sidx.at[b]], asrc.at[b],
                              gsem[b]).wait()
        pltpu.make_async_copy(asd_hbm.at[didx.at[b]], adst.at[b],
                              ssem[b]).wait()

    def _wait_scatter(b):
        pltpu.make_async_copy(evec.at[b], acc.at[didx.at[b]], csem[b]).wait()

    def _body(ch, b):
        nb = 1 - b
        _wait_gather(b)

        @pl.when(ch + 1 < NCH)
        def _prefetch():
            @pl.when(ch >= 1)
            def _drain():
                _wait_scatter(nb)
            _load_idx(ch + 1, nb)
            _issue(nb)

        bb = jnp.full((16,), b, jnp.int32)
        for g in range(KA // 16):
            rows = g * 16 + iota
            for h in range(H):
                sa = plsc.load_gather(
                    asrc, [bb, rows, jnp.full((16,), h, jnp.int32)])
                da = plsc.load_gather(
                    adst, [bb, rows, jnp.full((16,), 4 + h, jnp.int32)])
                al = sa + da
                al = jnp.where(al > 0, al, 0.2 * al)
                ev = jnp.exp(al)
                plsc.store_scatter(
                    evec, [bb, rows, jnp.full((16,), h, jnp.int32)], ev)
        pltpu.async_copy(evec.at[b], acc.at[didx.at[b]], csem[b], add=True)

    _load_idx(0, 0)
    _issue(0)

    def _pair(gg, _):
        for b in (0, 1):
            _body(gg * 2 + b, b)
        return 0
    lax.fori_loop(0, NCH // 2, _pair, 0)
    if NCH % 2 == 1:
        _body(NCH - 1, 0)
    _wait_scatter(0)
    _wait_scatter(1)

    plsc.subcore_barrier()
    pltpu.sync_copy(acc.at[pl.ds(rbase, RPT)], out_hbm.at[cid, pl.ds(rbase, RPT)])


def _mid_body(a0_ref, a1_ref, asd_ref, xglo_ref, xghi_ref, keep_ref, s16_ref,
              d16_ref, ns_ref, xslo_ref, xshi_ref):
    den = a0_ref[...] + a1_ref[...]
    rden = 1.0 / (den + 1e-16)
    dis = jnp.where(den > 0, lax.rsqrt(jnp.maximum(den, 1e-30)), 0.0)
    ns = asd_ref[...] * keep_ref[...]
    ns = ns + jnp.dot(rden, s16_ref[...], preferred_element_type=jnp.float32)
    ns = ns + jnp.dot(dis, d16_ref[...], preferred_element_type=jnp.float32)
    ns_ref[...] = ns
    dis1 = dis[:, 4:5]
    xslo_ref[...] = xglo_ref[...] * dis1
    xshi_ref[...] = xghi_ref[...] * dis1


def _tc_mid(a0, a1, asd, xglo, xghi, keep, s16, d16):
    grid = (NPAD // BLK,)
    return pl.pallas_call(
        _mid_body,
        grid=grid,
        in_specs=[
            pl.BlockSpec((BLK, 16), lambda i: (i, 0)),
            pl.BlockSpec((BLK, 16), lambda i: (i, 0)),
            pl.BlockSpec((BLK, 16), lambda i: (i, 0)),
            pl.BlockSpec((BLK, 64), lambda i: (i, 0)),
            pl.BlockSpec((BLK, 64), lambda i: (i, 0)),
            pl.BlockSpec((1, 16), lambda i: (0, 0)),
            pl.BlockSpec((16, 16), lambda i: (0, 0)),
            pl.BlockSpec((16, 16), lambda i: (0, 0)),
        ],
        out_specs=[
            pl.BlockSpec((BLK, 16), lambda i: (i, 0)),
            pl.BlockSpec((BLK, 64), lambda i: (i, 0)),
            pl.BlockSpec((BLK, 64), lambda i: (i, 0)),
        ],
        out_shape=[
            jax.ShapeDtypeStruct((NPAD, 16), jnp.float32),
            jax.ShapeDtypeStruct((NPAD, 64), jnp.float32),
            jax.ShapeDtypeStruct((NPAD, 64), jnp.float32),
        ],
    )(a0, a1, asd, xglo, xghi, keep, s16, d16)


def _sc_c_body(src_hbm, dst_hbm, ns_hbm, xwlo_hbm, xwhi_hbm, xglo_hbm, xghi_hbm,
               out_hbm, sidx, didx, ssml, dsml, wbuf, feats, zbuf, acc,
               gsem0, gsem1, ssem0, ssem1, dsem0, dsem1, csem0, csem1):
    gsem = (gsem0, gsem1)
    ssem = (ssem0, ssem1)
    dsem = (dsem0, dsem1)
    csem = (csem0, csem1)
    cid = lax.axis_index("c")
    sid = lax.axis_index("s")
    iota = lax.iota(jnp.int32, 16)
    zero16 = jnp.zeros((16,), jnp.float32)
    rbase = sid * RPT
    ebase0 = sid * EC

    def _zero_acc():
        def _zrow(r, _):
            for v in range(64 // 16):
                zbuf[r, pl.ds(v * 16, 16)] = zero16
            return 0
        lax.fori_loop(0, RPT // 4, _zrow, 0)
        for q in range(4):
            pltpu.sync_copy(zbuf, acc.at[pl.ds(rbase + q * (RPT // 4), RPT // 4)])

    NCH = EC // KC

    def _load_idx(ch, b):
        base = pl.multiple_of(ebase0 + ch * KC, 8)
        pltpu.sync_copy(src_hbm.at[pl.ds(base, KC)], sidx.at[b])
        pltpu.sync_copy(dst_hbm.at[pl.ds(base, KC)], didx.at[b])

    def _pipe(tab_hbm, gat_hf):
        is_gat = gat_hf is not None

        def _issue(b):
            pltpu.async_copy(tab_hbm.at[sidx.at[b]], feats.at[b], gsem[b])
            if is_gat:
                pltpu.async_copy(ns_hbm.at[sidx.at[b]], ssml.at[b], ssem[b])
                pltpu.async_copy(ns_hbm.at[didx.at[b]], dsml.at[b], dsem[b])

        def _wait_gather(b):
            pltpu.make_async_copy(tab_hbm.at[sidx.at[b]], feats.at[b],
                                  gsem[b]).wait()
            if is_gat:
                pltpu.make_async_copy(ns_hbm.at[sidx.at[b]], ssml.at[b],
                                      ssem[b]).wait()
                pltpu.make_async_copy(ns_hbm.at[didx.at[b]], dsml.at[b],
                                      dsem[b]).wait()

        def _wait_scatter(b):
            pltpu.make_async_copy(feats.at[b], acc.at[didx.at[b]],
                                  csem[b]).wait()

        _load_idx(0, 0)
        _issue(0)

        def _pair(gg, _):
            for b in (0, 1):
                ch = gg * 2 + b
                nb = 1 - b
                _wait_gather(b)

                @pl.when(ch + 1 < NCH)
                def _prefetch():
                    @pl.when(ch >= 1)
                    def _drain():
                        _wait_scatter(nb)
                    _load_idx(ch + 1, nb)
                    _issue(nb)

                if is_gat:
                    bb = jnp.full((16,), b, jnp.int32)
                    for g in range(KC // 16):
                        rows = g * 16 + iota
                        for h in (2 * gat_hf, 2 * gat_hf + 1):
                            sa = plsc.load_gather(
                                ssml, [bb, rows, jnp.full((16,), h, jnp.int32)])
                            da = plsc.load_gather(
                                dsml, [bb, rows, jnp.full((16,), 4 + h, jnp.int32)])
                            rd = plsc.load_gather(
                                dsml, [bb, rows, jnp.full((16,), 8 + h, jnp.int32)])
                            al = sa + da
                            al = jnp.where(al > 0, al, 0.2 * al)
                            w = jnp.exp(al) * rd
                            plsc.store_scatter(
                                wbuf, [rows, jnp.full((16,), h, jnp.int32)], w)

                    def _edge(j4, _):
                        for u in range(4):
                            j = j4 * 4 + u
                            j16 = jnp.full((16,), j, jnp.int32)
                            for hh in range(2):
                                wsp = plsc.load_gather(
                                    wbuf,
                                    [j16, jnp.full((16,), 2 * gat_hf + hh,
                                                   jnp.int32)])
                                for half in range(2):
                                    v = 2 * hh + half
                                    feats[b, j, pl.ds(v * 16, 16)] = (
                                        feats[b, j, pl.ds(v * 16, 16)] * wsp)
                        return 0
                    lax.fori_loop(0, KC // 4, _edge, 0)

                pltpu.async_copy(feats.at[b], acc.at[didx.at[b]], csem[b],
                                 add=True)
            return 0
        lax.fori_loop(0, NCH // 2, _pair, 0)
        _wait_scatter(0)
        _wait_scatter(1)

    for p in range(2):
        _zero_acc()
        plsc.subcore_barrier()

        if p == 0:
            @pl.when(cid == 0)
            def _gat_lo():
                _pipe(xwlo_hbm, 0)

            @pl.when(cid == 1)
            def _gat_hi():
                _pipe(xwhi_hbm, 1)
        else:
            @pl.when(cid == 0)
            def _gcn_lo():
                _pipe(xglo_hbm, None)

            @pl.when(cid == 1)
            def _gcn_hi():
                _pipe(xghi_hbm, None)

        plsc.subcore_barrier()
        pltpu.sync_copy(acc.at[pl.ds(rbase, RPT)],
                        out_hbm.at[p, cid, pl.ds(rbase, RPT)])


def _post_body(gatlo_ref, gathi_ref, gcnlo_ref, gcnhi_ref, ns_ref, x_ref,
               bgat_ref, bgcn_ref, wga_ref, wgb_ref, bgate_ref, gamma_ref,
               beta_ref, o_ref):
    dis1 = ns_ref[:, 12:13]
    gat = jnp.concatenate([gatlo_ref[...], gathi_ref[...]], axis=1) + bgat_ref[...]
    gcn = (jnp.concatenate([gcnlo_ref[...], gcnhi_ref[...]], axis=1) * dis1
           + bgcn_ref[...])
    lg = (jnp.dot(gat, wga_ref[...], preferred_element_type=jnp.float32)
          + jnp.dot(gcn, wgb_ref[...], preferred_element_type=jnp.float32)
          + bgate_ref[...])
    m = jnp.max(lg, axis=-1, keepdims=True)
    eg = jnp.exp(lg - m)
    sm = eg / jnp.sum(eg, axis=-1, keepdims=True)
    out = sm[:, 0:1] * gat + sm[:, 1:2] * gcn
    y = out + x_ref[...]
    mu = jnp.mean(y, axis=-1, keepdims=True)
    yc = y - mu
    var = jnp.mean(yc * yc, axis=-1, keepdims=True)
    o_ref[...] = gamma_ref[...] * yc * lax.rsqrt(var + 1e-5) + beta_ref[...]


def _tc_post(gatlo, gathi, gcnlo, gcnhi, nsmall, xpad, b_gat, b_gcn, wga, wgb,
             b_gate, gamma, beta):
    grid = (NPAD // BLK,)
    return pl.pallas_call(
        _post_body,
        grid=grid,
        in_specs=[
            pl.BlockSpec((BLK, 64), lambda i: (i, 0)),
            pl.BlockSpec((BLK, 64), lambda i: (i, 0)),
            pl.BlockSpec((BLK, 64), lambda i: (i, 0)),
            pl.BlockSpec((BLK, 64), lambda i: (i, 0)),
            pl.BlockSpec((BLK, 16), lambda i: (i, 0)),
            pl.BlockSpec((BLK, D), lambda i: (i, 0)),
            pl.BlockSpec((1, D), lambda i: (0, 0)),
            pl.BlockSpec((1, D), lambda i: (0, 0)),
            pl.BlockSpec((D, 2), lambda i: (0, 0)),
            pl.BlockSpec((D, 2), lambda i: (0, 0)),
            pl.BlockSpec((1, 2), lambda i: (0, 0)),
            pl.BlockSpec((1, D), lambda i: (0, 0)),
            pl.BlockSpec((1, D), lambda i: (0, 0)),
        ],
        out_specs=pl.BlockSpec((BLK, D), lambda i: (i, 0)),
        out_shape=jax.ShapeDtypeStruct((NPAD, D), jnp.float32),
    )(gatlo, gathi, gcnlo, gcnhi, nsmall, xpad, b_gat, b_gcn, wga, wgb,
      b_gate, gamma, beta)


def _sc_kernels():
    if "a" not in _SC_CACHE:
        mesh = plsc.VectorSubcoreMesh(core_axis_name="c", subcore_axis_name="s")
        _SC_CACHE["a"] = pl.kernel(
            _sc_a_body,
            mesh=mesh,
            compiler_params=pltpu.CompilerParams(
                needs_layout_passes=False, use_tc_tiling_on_sc=False),
            out_type=jax.ShapeDtypeStruct((2, NPAD, 16), jnp.float32),
            scratch_types=[
                pltpu.VMEM((2, KA), jnp.int32),
                pltpu.VMEM((2, KA), jnp.int32),
                pltpu.VMEM((2, KA, 16), jnp.float32),
                pltpu.VMEM((2, KA, 16), jnp.float32),
                pltpu.VMEM((2, KA, 16), jnp.float32),
                pltpu.VMEM((RPT, 16), jnp.float32),
                pltpu.VMEM_SHARED((NPAD, 16), jnp.float32),
                pltpu.SemaphoreType.DMA,
                pltpu.SemaphoreType.DMA,
                pltpu.SemaphoreType.DMA,
                pltpu.SemaphoreType.DMA,
                pltpu.SemaphoreType.DMA,
                pltpu.SemaphoreType.DMA,
            ],
        )
        _SC_CACHE["c"] = pl.kernel(
            _sc_c_body,
            mesh=mesh,
            compiler_params=pltpu.CompilerParams(
                needs_layout_passes=False, use_tc_tiling_on_sc=False),
            out_type=jax.ShapeDtypeStruct((2, 2, NPAD, 64), jnp.float32),
            scratch_types=[
                pltpu.VMEM((2, KC), jnp.int32),
                pltpu.VMEM((2, KC), jnp.int32),
                pltpu.VMEM((2, KC, 16), jnp.float32),
                pltpu.VMEM((2, KC, 16), jnp.float32),
                pltpu.VMEM((KC, 16), jnp.float32),
                pltpu.VMEM((2, KC, 64), jnp.float32),
                pltpu.VMEM((RPT // 4, 64), jnp.float32),
                pltpu.VMEM_SHARED((NPAD, 64), jnp.float32),
                pltpu.SemaphoreType.DMA,
                pltpu.SemaphoreType.DMA,
                pltpu.SemaphoreType.DMA,
                pltpu.SemaphoreType.DMA,
                pltpu.SemaphoreType.DMA,
                pltpu.SemaphoreType.DMA,
                pltpu.SemaphoreType.DMA,
                pltpu.SemaphoreType.DMA,
            ],
        )
    return _SC_CACHE["a"], _SC_CACHE["c"]


def kernel(x, edge_index, W_gat, att_src, att_dst, b_gat, W_gcn, b_gcn,
           W_gate, b_gate, gamma, beta):
    xpad = jnp.pad(x, ((0, NPAD - N), (0, 0)))
    loops = jnp.arange(N, dtype=jnp.int32)
    src = jnp.concatenate([edge_index[0], loops,
                           jnp.zeros((ETPAD - ET,), jnp.int32)])
    dst = jnp.concatenate([edge_index[1], loops,
                           jnp.full((ETPAD - ET,), N, jnp.int32)])

    eyeH = jnp.eye(H, dtype=jnp.float32)
    A_src = (att_src[:, :, None] * eyeH[:, None, :]).reshape(D, H)
    A_dst = (att_dst[:, :, None] * eyeH[:, None, :]).reshape(D, H)
    A_comb = jnp.concatenate([A_src, A_dst, jnp.zeros((D, 8), jnp.float32)], axis=1)

    xwlo, xwhi, xglo, xghi, asd = _tc_pre(xpad, W_gat, W_gcn, A_comb)

    sc_a, sc_c = _sc_kernels()
    accA = sc_a(src, dst, asd)

    col = jnp.arange(16)
    keep = (col < 8).astype(jnp.float32)[None, :]
    s16 = ((col[:, None] + 8 == col[None, :]) & (col[:, None] < 4)).astype(jnp.float32)
    d16 = ((col[:, None] == 4) & (col[None, :] == 12)).astype(jnp.float32)
    nsmall, xslo, xshi = _tc_mid(accA[0], accA[1], asd, xglo, xghi,
                                 keep, s16, d16)

    raw = sc_c(src, dst, nsmall, xwlo, xwhi, xslo, xshi)

    y = _tc_post(raw[0, 0], raw[0, 1], raw[1, 0], raw[1, 1], nsmall, xpad,
                 b_gat[None, :], b_gcn[None, :], W_gate[:D], W_gate[D:],
                 b_gate[None, :], gamma[None, :], beta[None, :])
    return y[:N]
